# Initial kernel scaffold; baseline (speedup 1.0000x reference)
import jax
import jax.numpy as jnp
from jax.experimental import pallas as pl

H = 64
W = 64
N = 4096
NEAR = 0.4
FAR = 1000.0
BT = 48.0
C = 128
PB = 8
NPIX = PB * W


def _prep_body(pts_ref, feats_ref, maskf_ref, cam_ref, attrs_ref):
    fx = cam_ref[0, 0]
    fy = cam_ref[0, 1]
    cx = cam_ref[0, 2]
    cy = cam_ref[0, 3]
    qw = cam_ref[0, 4]
    qx = cam_ref[0, 5]
    qy = cam_ref[0, 6]
    qz = cam_ref[0, 7]
    tx = cam_ref[0, 8]
    ty = cam_ref[0, 9]
    tz = cam_ref[0, 10]
    qn = jax.lax.rsqrt(qw * qw + qx * qx + qy * qy + qz * qz)
    w = qw * qn
    x = qx * qn
    y = qy * qn
    z_ = qz * qn
    r00 = 1 - 2 * (y * y + z_ * z_)
    r01 = 2 * (x * y - w * z_)
    r02 = 2 * (x * z_ + w * y)
    r10 = 2 * (x * y + w * z_)
    r11 = 1 - 2 * (x * x + z_ * z_)
    r12 = 2 * (y * z_ - w * x)
    r20 = 2 * (x * z_ - w * y)
    r21 = 2 * (y * z_ + w * x)
    r22 = 1 - 2 * (x * x + y * y)
    R = ((r00, r01, r02), (r10, r11, r12), (r20, r21, r22))

    px = pts_ref[0]
    py = pts_ref[1]
    pz = pts_ref[2]
    xc = r00 * px + r01 * py + r02 * pz + tx
    yc = r10 * px + r11 * py + r12 * pz + ty
    zc = r20 * px + r21 * py + r22 * pz + tz
    zcl = jnp.where(jnp.abs(zc) < 1e-6, 1e-6, zc)
    u = fx * xc / zcl + cx
    v = fy * yc / zcl + cy

    f0 = feats_ref[0]
    f1 = feats_ref[1]
    f2 = feats_ref[2]
    f3 = feats_ref[3]
    gqn = jax.lax.rsqrt(f0 * f0 + f1 * f1 + f2 * f2 + f3 * f3)
    gw = f0 * gqn
    gx = f1 * gqn
    gy = f2 * gqn
    gz = f3 * gqn
    g00 = 1 - 2 * (gy * gy + gz * gz)
    g01 = 2 * (gx * gy - gw * gz)
    g02 = 2 * (gx * gz + gw * gy)
    g10 = 2 * (gx * gy + gw * gz)
    g11 = 1 - 2 * (gx * gx + gz * gz)
    g12 = 2 * (gy * gz - gw * gx)
    g20 = 2 * (gx * gz - gw * gy)
    g21 = 2 * (gy * gz + gw * gx)
    g22 = 1 - 2 * (gx * gx + gy * gy)
    G = ((g00, g01, g02), (g10, g11, g12), (g20, g21, g22))

    s0 = jnp.exp(feats_ref[4])
    s1 = jnp.exp(feats_ref[5])
    s2 = jnp.exp(feats_ref[6])
    sq = (s0 * s0, s1 * s1, s2 * s2)
    alpha = jax.nn.sigmoid(feats_ref[7])
    col_r = jnp.clip(0.5 + 0.28209479177 * feats_ref[8], 0.0, 1.0)
    col_g = jnp.clip(0.5 + 0.28209479177 * feats_ref[9], 0.0, 1.0)
    col_b = jnp.clip(0.5 + 0.28209479177 * feats_ref[10], 0.0, 1.0)

    M = [[R[a][0] * G[0][b] + R[a][1] * G[1][b] + R[a][2] * G[2][b]
          for b in range(3)] for a in range(3)]
    j00 = fx / zcl
    j02 = -fx * xc / (zcl * zcl)
    j11 = fy / zcl
    j12 = -fy * yc / (zcl * zcl)
    k0 = [j00 * M[0][b] + j02 * M[2][b] for b in range(3)]
    k1 = [j11 * M[1][b] + j12 * M[2][b] for b in range(3)]
    a = sq[0] * k0[0] * k0[0] + sq[1] * k0[1] * k0[1] + sq[2] * k0[2] * k0[2] + 0.3
    d = sq[0] * k1[0] * k1[0] + sq[1] * k1[1] * k1[1] + sq[2] * k1[2] * k1[2] + 0.3
    bb = sq[0] * k0[0] * k1[0] + sq[1] * k0[1] * k1[1] + sq[2] * k0[2] * k1[2]
    det = jnp.maximum(a * d - bb * bb, 1e-9)
    inv_a = d / det
    inv_b = -bb / det
    inv_d = a / det

    valid = ((zc > NEAR) & (zc < FAR)
             & (u >= -BT) & (u < W + BT) & (v >= -BT) & (v < H + BT)
             & (maskf_ref[0] < 0.5))
    alpha = jnp.where(valid, alpha, 0.0)

    attrs_ref[0] = u
    attrs_ref[1] = v
    attrs_ref[2] = inv_a
    attrs_ref[3] = inv_b
    attrs_ref[4] = inv_d
    attrs_ref[5] = alpha
    attrs_ref[6] = col_r
    attrs_ref[7] = col_g
    attrs_ref[8] = col_b
    attrs_ref[9] = zc
    zero = jnp.zeros_like(u)
    for k in range(10, 16):
        attrs_ref[k] = zero


def _rank_body(zrow_ref, zcol_ref, rank_ref):
    zrow = zrow_ref[0:1, :]
    irow = jax.lax.broadcasted_iota(jnp.float32, (1, N), 1)
    acc = jnp.zeros((1, N), jnp.float32)
    for c in range(N // C):
        zc = zcol_ref[pl.ds(c * C, C), 0:1]
        jcol = jax.lax.broadcasted_iota(jnp.float32, (C, 1), 0) + float(c * C)
        lt = zc < zrow
        eq = (zc == zrow) & (jcol < irow)
        cmp = jnp.where(lt | eq, 1.0, 0.0)
        acc = acc + jnp.sum(cmp, axis=0, keepdims=True)
    rank_ref[0:1, :] = acc


def _permute_body(attrs_ref, rankcol_ref, out_ref):
    b = pl.program_id(0)
    rk = rankcol_ref[:, 0:1]
    rr = (jax.lax.broadcasted_iota(jnp.float32, (1, 512), 1)
          + (b * 512).astype(jnp.float32))
    oh = jnp.where(rk == rr, 1.0, 0.0)
    out_ref[...] = jnp.dot(attrs_ref[...], oh,
                           preferred_element_type=jnp.float32)


def _render_body(st_ref, cz_ref, img_ref, dep_ref, acc_ref):
    pid = pl.program_id(0)
    flat = jax.lax.broadcasted_iota(jnp.float32, (NPIX, 1), 0)
    pxc = jnp.mod(flat, float(W)) + 0.5
    pyc = jnp.floor(flat / float(W)) + (pid * PB).astype(jnp.float32) + 0.5

    jj = jax.lax.broadcasted_iota(jnp.float32, (C, C), 0)
    kk = jax.lax.broadcasted_iota(jnp.float32, (C, C), 1)
    sut = jnp.where(jj < kk, 1.0, 0.0)

    accum = jnp.zeros((NPIX, 8), jnp.float32)
    carry = jnp.zeros((NPIX, 1), jnp.float32)
    for c in range(N // C):
        sl = pl.ds(c * C, C)
        u = st_ref[0:1, sl]
        v = st_ref[1:2, sl]
        ia = st_ref[2:3, sl]
        ib = st_ref[3:4, sl]
        idd = st_ref[4:5, sl]
        al = st_ref[5:6, sl]
        du = pxc - u
        dv = pyc - v
        power = -0.5 * (ia * du * du + idd * dv * dv) - ib * du * dv
        g = jnp.exp(jnp.minimum(power, 0.0))
        ai = jnp.minimum(al * g, 0.99)
        ai = jnp.where(ai > jnp.float32(1.0 / 255.0), ai, 0.0)
        logt = jnp.log(1.0 - ai)
        ecs = jnp.dot(logt, sut, preferred_element_type=jnp.float32)
        tprev = jnp.exp(carry + ecs)
        wgt = ai * tprev
        accum = accum + jnp.dot(wgt, cz_ref[sl, :],
                                preferred_element_type=jnp.float32)
        carry = carry + jnp.sum(logt, axis=1, keepdims=True)

    img_ref[...] = accum[:, 0:3]
    wsum = accum[:, 4:5]
    dep_ref[...] = accum[:, 3:4] / jnp.maximum(wsum, 1e-6)
    acc_ref[...] = wsum


@jax.jit
def _pipeline(pts3, feats11, maskf, cam):
    attrs = pl.pallas_call(
        _prep_body,
        out_shape=jax.ShapeDtypeStruct((16, 32, 128), jnp.float32),
    )(pts3, feats11, maskf, cam)
    attrs_t = attrs.reshape(16, N)
    z_row = attrs_t[9:10, :]
    z_col = z_row.T

    rank_row = pl.pallas_call(
        _rank_body,
        out_shape=jax.ShapeDtypeStruct((1, N), jnp.float32),
    )(z_row, z_col)
    rank_col = rank_row.T

    sorted_t = pl.pallas_call(
        _permute_body,
        grid=(N // 512,),
        in_specs=[
            pl.BlockSpec((16, N), lambda b: (0, 0)),
            pl.BlockSpec((N, 1), lambda b: (0, 0)),
        ],
        out_specs=pl.BlockSpec((16, 512), lambda b: (0, b)),
        out_shape=jax.ShapeDtypeStruct((16, N), jnp.float32),
    )(attrs_t, rank_col)

    colorz = jnp.concatenate(
        [sorted_t[6:10, :],
         jnp.ones((1, N), jnp.float32),
         jnp.zeros((3, N), jnp.float32)], axis=0).T

    img, dep, acc = pl.pallas_call(
        _render_body,
        grid=(H // PB,),
        in_specs=[
            pl.BlockSpec((16, N), lambda b: (0, 0)),
            pl.BlockSpec((N, 8), lambda b: (0, 0)),
        ],
        out_specs=[
            pl.BlockSpec((NPIX, 3), lambda b: (b, 0)),
            pl.BlockSpec((NPIX, 1), lambda b: (b, 0)),
            pl.BlockSpec((NPIX, 1), lambda b: (b, 0)),
        ],
        out_shape=[
            jax.ShapeDtypeStruct((H * W, 3), jnp.float32),
            jax.ShapeDtypeStruct((H * W, 1), jnp.float32),
            jax.ShapeDtypeStruct((H * W, 1), jnp.float32),
        ],
    )(sorted_t, colorz)
    return (img.reshape(H, W, 3), dep.reshape(H, W), acc.reshape(H, W))


def kernel(pointcloud, pointcloud_features, point_invalid_mask,
           point_object_id, camera_intrinsics, q_camera_pointcloud,
           t_camera_pointcloud):
    del point_object_id
    pts3 = pointcloud.T.reshape(3, 32, 128)
    feat_cols = jnp.stack(
        [pointcloud_features[:, j]
         for j in (0, 1, 2, 3, 4, 5, 6, 7, 8, 24, 40)],
        axis=0).reshape(11, 32, 128)
    maskf = point_invalid_mask.astype(jnp.float32).reshape(1, 32, 128)
    cam = jnp.concatenate([
        jnp.stack([camera_intrinsics[0, 0], camera_intrinsics[1, 1],
                   camera_intrinsics[0, 2], camera_intrinsics[1, 2]]),
        q_camera_pointcloud[0],
        t_camera_pointcloud[0],
        jnp.zeros((5,), jnp.float32),
    ]).reshape(1, 16)
    return _pipeline(pts3, feat_cols, maskf, cam)


# speedup vs baseline: 15.0842x; 15.0842x over previous
import jax
import jax.numpy as jnp
from jax.experimental import pallas as pl

H = 64
W = 64
N = 4096
NEAR = 0.4
FAR = 1000.0
BT = 48.0
C = 128
PB = 8
NPIX = PB * W


def _prep_body(pts_ref, feats_ref, maskf_ref, cam_ref, attrs_ref):
    fx = cam_ref[0, 0]
    fy = cam_ref[0, 1]
    cx = cam_ref[0, 2]
    cy = cam_ref[0, 3]
    qw = cam_ref[0, 4]
    qx = cam_ref[0, 5]
    qy = cam_ref[0, 6]
    qz = cam_ref[0, 7]
    tx = cam_ref[0, 8]
    ty = cam_ref[0, 9]
    tz = cam_ref[0, 10]
    qn = jax.lax.rsqrt(qw * qw + qx * qx + qy * qy + qz * qz)
    w = qw * qn
    x = qx * qn
    y = qy * qn
    z_ = qz * qn
    r00 = 1 - 2 * (y * y + z_ * z_)
    r01 = 2 * (x * y - w * z_)
    r02 = 2 * (x * z_ + w * y)
    r10 = 2 * (x * y + w * z_)
    r11 = 1 - 2 * (x * x + z_ * z_)
    r12 = 2 * (y * z_ - w * x)
    r20 = 2 * (x * z_ - w * y)
    r21 = 2 * (y * z_ + w * x)
    r22 = 1 - 2 * (x * x + y * y)
    R = ((r00, r01, r02), (r10, r11, r12), (r20, r21, r22))

    px = pts_ref[0]
    py = pts_ref[1]
    pz = pts_ref[2]
    xc = r00 * px + r01 * py + r02 * pz + tx
    yc = r10 * px + r11 * py + r12 * pz + ty
    zc = r20 * px + r21 * py + r22 * pz + tz
    zcl = jnp.where(jnp.abs(zc) < 1e-6, 1e-6, zc)
    u = fx * xc / zcl + cx
    v = fy * yc / zcl + cy

    f0 = feats_ref[0]
    f1 = feats_ref[1]
    f2 = feats_ref[2]
    f3 = feats_ref[3]
    gqn = jax.lax.rsqrt(f0 * f0 + f1 * f1 + f2 * f2 + f3 * f3)
    gw = f0 * gqn
    gx = f1 * gqn
    gy = f2 * gqn
    gz = f3 * gqn
    g00 = 1 - 2 * (gy * gy + gz * gz)
    g01 = 2 * (gx * gy - gw * gz)
    g02 = 2 * (gx * gz + gw * gy)
    g10 = 2 * (gx * gy + gw * gz)
    g11 = 1 - 2 * (gx * gx + gz * gz)
    g12 = 2 * (gy * gz - gw * gx)
    g20 = 2 * (gx * gz - gw * gy)
    g21 = 2 * (gy * gz + gw * gx)
    g22 = 1 - 2 * (gx * gx + gy * gy)
    G = ((g00, g01, g02), (g10, g11, g12), (g20, g21, g22))

    s0 = jnp.exp(feats_ref[4])
    s1 = jnp.exp(feats_ref[5])
    s2 = jnp.exp(feats_ref[6])
    sq = (s0 * s0, s1 * s1, s2 * s2)
    alpha = jax.nn.sigmoid(feats_ref[7])
    col_r = jnp.clip(0.5 + 0.28209479177 * feats_ref[8], 0.0, 1.0)
    col_g = jnp.clip(0.5 + 0.28209479177 * feats_ref[9], 0.0, 1.0)
    col_b = jnp.clip(0.5 + 0.28209479177 * feats_ref[10], 0.0, 1.0)

    M = [[R[a][0] * G[0][b] + R[a][1] * G[1][b] + R[a][2] * G[2][b]
          for b in range(3)] for a in range(3)]
    j00 = fx / zcl
    j02 = -fx * xc / (zcl * zcl)
    j11 = fy / zcl
    j12 = -fy * yc / (zcl * zcl)
    k0 = [j00 * M[0][b] + j02 * M[2][b] for b in range(3)]
    k1 = [j11 * M[1][b] + j12 * M[2][b] for b in range(3)]
    a = sq[0] * k0[0] * k0[0] + sq[1] * k0[1] * k0[1] + sq[2] * k0[2] * k0[2] + 0.3
    d = sq[0] * k1[0] * k1[0] + sq[1] * k1[1] * k1[1] + sq[2] * k1[2] * k1[2] + 0.3
    bb = sq[0] * k0[0] * k1[0] + sq[1] * k0[1] * k1[1] + sq[2] * k0[2] * k1[2]
    det = jnp.maximum(a * d - bb * bb, 1e-9)
    inv_a = d / det
    inv_b = -bb / det
    inv_d = a / det

    valid = ((zc > NEAR) & (zc < FAR)
             & (u >= -BT) & (u < W + BT) & (v >= -BT) & (v < H + BT)
             & (maskf_ref[0] < 0.5))
    alpha = jnp.where(valid, alpha, 0.0)

    attrs_ref[0] = u
    attrs_ref[1] = v
    attrs_ref[2] = inv_a
    attrs_ref[3] = inv_b
    attrs_ref[4] = inv_d
    attrs_ref[5] = alpha
    attrs_ref[6] = col_r
    attrs_ref[7] = col_g
    attrs_ref[8] = col_b
    attrs_ref[9] = zc
    zero = jnp.zeros_like(u)
    for k in range(10, 16):
        attrs_ref[k] = zero


def _rank_body(zrow_ref, zcol_ref, rank_ref):
    zrow = zrow_ref[0:1, :]
    irow = jax.lax.broadcasted_iota(jnp.int32, (1, N), 1).astype(jnp.float32)
    acc = jnp.zeros((1, N), jnp.float32)
    for c in range(N // C):
        zc = zcol_ref[pl.ds(c * C, C), 0:1]
        jcol = (jax.lax.broadcasted_iota(jnp.int32, (C, 1), 0)
                .astype(jnp.float32) + float(c * C))
        lt = zc < zrow
        eq = (zc == zrow) & (jcol < irow)
        cmp = jnp.where(lt | eq, 1.0, 0.0)
        acc = acc + jnp.sum(cmp, axis=0, keepdims=True)
    rank_ref[0:1, :] = acc


def _permute_body(attrs_ref, rankcol_ref, out_ref):
    b = pl.program_id(0)
    rk = rankcol_ref[:, 0:1]
    rr = (jax.lax.broadcasted_iota(jnp.int32, (1, 512), 1) + b * 512
          ).astype(jnp.float32)
    oh = jnp.where(rk == rr, 1.0, 0.0)
    out_ref[...] = jnp.dot(attrs_ref[...], oh,
                           preferred_element_type=jnp.float32)


def _render_body(st_ref, cz_ref, img_ref, dep_ref, acc_ref):
    pid = pl.program_id(0)
    flat = jax.lax.broadcasted_iota(jnp.int32, (NPIX, 1), 0)
    pxc = jnp.mod(flat, W).astype(jnp.float32) + 0.5
    pyc = (flat // W + pid * PB).astype(jnp.float32) + 0.5

    jj = jax.lax.broadcasted_iota(jnp.int32, (C, C), 0)
    kk = jax.lax.broadcasted_iota(jnp.int32, (C, C), 1)
    sut = jnp.where(jj < kk, 1.0, 0.0)

    accum = jnp.zeros((NPIX, 8), jnp.float32)
    carry = jnp.zeros((NPIX, 1), jnp.float32)
    for c in range(N // C):
        sl = pl.ds(c * C, C)
        u = st_ref[0:1, sl]
        v = st_ref[1:2, sl]
        ia = st_ref[2:3, sl]
        ib = st_ref[3:4, sl]
        idd = st_ref[4:5, sl]
        al = st_ref[5:6, sl]
        du = pxc - u
        dv = pyc - v
        power = -0.5 * (ia * du * du + idd * dv * dv) - ib * du * dv
        g = jnp.exp(jnp.minimum(power, 0.0))
        ai = jnp.minimum(al * g, 0.99)
        ai = jnp.where(ai > jnp.float32(1.0 / 255.0), ai, 0.0)
        logt = jnp.log(1.0 - ai)
        ecs = jnp.dot(logt, sut, preferred_element_type=jnp.float32)
        tprev = jnp.exp(carry + ecs)
        wgt = ai * tprev
        accum = accum + jnp.dot(wgt, cz_ref[sl, :],
                                preferred_element_type=jnp.float32)
        carry = carry + jnp.sum(logt, axis=1, keepdims=True)

    img_ref[...] = accum[:, 0:3]
    wsum = accum[:, 4:5]
    dep_ref[...] = accum[:, 3:4] / jnp.maximum(wsum, 1e-6)
    acc_ref[...] = wsum


@jax.jit
def _pipeline(pts3, feats11, maskf, cam):
    attrs = pl.pallas_call(
        _prep_body,
        out_shape=jax.ShapeDtypeStruct((16, 32, 128), jnp.float32),
    )(pts3, feats11, maskf, cam)
    attrs_t = attrs.reshape(16, N)
    z_row = attrs_t[9:10, :]
    z_col = z_row.T

    rank_row = pl.pallas_call(
        _rank_body,
        out_shape=jax.ShapeDtypeStruct((1, N), jnp.float32),
    )(z_row, z_col)
    rank_col = rank_row.T

    sorted_t = pl.pallas_call(
        _permute_body,
        grid=(N // 512,),
        in_specs=[
            pl.BlockSpec((16, N), lambda b: (0, 0)),
            pl.BlockSpec((N, 1), lambda b: (0, 0)),
        ],
        out_specs=pl.BlockSpec((16, 512), lambda b: (0, b)),
        out_shape=jax.ShapeDtypeStruct((16, N), jnp.float32),
    )(attrs_t, rank_col)

    colorz = jnp.concatenate(
        [sorted_t[6:10, :],
         jnp.ones((1, N), jnp.float32),
         jnp.zeros((3, N), jnp.float32)], axis=0).T

    img, dep, acc = pl.pallas_call(
        _render_body,
        grid=(H // PB,),
        in_specs=[
            pl.BlockSpec((16, N), lambda b: (0, 0)),
            pl.BlockSpec((N, 8), lambda b: (0, 0)),
        ],
        out_specs=[
            pl.BlockSpec((NPIX, 3), lambda b: (b, 0)),
            pl.BlockSpec((NPIX, 1), lambda b: (b, 0)),
            pl.BlockSpec((NPIX, 1), lambda b: (b, 0)),
        ],
        out_shape=[
            jax.ShapeDtypeStruct((H * W, 3), jnp.float32),
            jax.ShapeDtypeStruct((H * W, 1), jnp.float32),
            jax.ShapeDtypeStruct((H * W, 1), jnp.float32),
        ],
    )(sorted_t, colorz)
    return (img.reshape(H, W, 3), dep.reshape(H, W), acc.reshape(H, W))


def kernel(pointcloud, pointcloud_features, point_invalid_mask,
           point_object_id, camera_intrinsics, q_camera_pointcloud,
           t_camera_pointcloud):
    del point_object_id
    pts3 = pointcloud.T.reshape(3, 32, 128)
    feat_cols = jnp.stack(
        [pointcloud_features[:, j]
         for j in (0, 1, 2, 3, 4, 5, 6, 7, 8, 24, 40)],
        axis=0).reshape(11, 32, 128)
    maskf = point_invalid_mask.astype(jnp.float32).reshape(1, 32, 128)
    cam = jnp.concatenate([
        jnp.stack([camera_intrinsics[0, 0], camera_intrinsics[1, 1],
                   camera_intrinsics[0, 2], camera_intrinsics[1, 2]]),
        q_camera_pointcloud[0],
        t_camera_pointcloud[0],
        jnp.zeros((5,), jnp.float32),
    ]).reshape(1, 16)
    return _pipeline(pts3, feat_cols, maskf, cam)


# speedup vs baseline: 17.9058x; 1.1871x over previous
import functools

import jax
import jax.numpy as jnp
from jax import lax
from jax.experimental import pallas as pl
from jax.experimental.pallas import tpu as pltpu
from jax.experimental.pallas import tpu_sc as plsc

H = 64
W = 64
N = 4096
NEAR = 0.4
FAR = 1000.0
BT = 48.0
C = 128
TILE = 16
TX = W // TILE
TY = H // TILE
NT = TX * TY
NWORK = 32
HALF = N // 2
TPIX = TILE * TILE


def _prep_body(pts_ref, feats_ref, maskf_ref, cam_ref, attrs_ref):
    fx = cam_ref[0, 0]
    fy = cam_ref[0, 1]
    cx = cam_ref[0, 2]
    cy = cam_ref[0, 3]
    qw = cam_ref[0, 4]
    qx = cam_ref[0, 5]
    qy = cam_ref[0, 6]
    qz = cam_ref[0, 7]
    tx = cam_ref[0, 8]
    ty = cam_ref[0, 9]
    tz = cam_ref[0, 10]
    qn = lax.rsqrt(qw * qw + qx * qx + qy * qy + qz * qz)
    w = qw * qn
    x = qx * qn
    y = qy * qn
    z_ = qz * qn
    r00 = 1 - 2 * (y * y + z_ * z_)
    r01 = 2 * (x * y - w * z_)
    r02 = 2 * (x * z_ + w * y)
    r10 = 2 * (x * y + w * z_)
    r11 = 1 - 2 * (x * x + z_ * z_)
    r12 = 2 * (y * z_ - w * x)
    r20 = 2 * (x * z_ - w * y)
    r21 = 2 * (y * z_ + w * x)
    r22 = 1 - 2 * (x * x + y * y)
    R = ((r00, r01, r02), (r10, r11, r12), (r20, r21, r22))

    px = pts_ref[0]
    py = pts_ref[1]
    pz = pts_ref[2]
    xc = r00 * px + r01 * py + r02 * pz + tx
    yc = r10 * px + r11 * py + r12 * pz + ty
    zc = r20 * px + r21 * py + r22 * pz + tz
    zcl = jnp.where(jnp.abs(zc) < 1e-6, 1e-6, zc)
    u = fx * xc / zcl + cx
    v = fy * yc / zcl + cy

    f0 = feats_ref[0]
    f1 = feats_ref[1]
    f2 = feats_ref[2]
    f3 = feats_ref[3]
    gqn = lax.rsqrt(f0 * f0 + f1 * f1 + f2 * f2 + f3 * f3)
    gw = f0 * gqn
    gx = f1 * gqn
    gy = f2 * gqn
    gz = f3 * gqn
    g00 = 1 - 2 * (gy * gy + gz * gz)
    g01 = 2 * (gx * gy - gw * gz)
    g02 = 2 * (gx * gz + gw * gy)
    g10 = 2 * (gx * gy + gw * gz)
    g11 = 1 - 2 * (gx * gx + gz * gz)
    g12 = 2 * (gy * gz - gw * gx)
    g20 = 2 * (gx * gz - gw * gy)
    g21 = 2 * (gy * gz + gw * gx)
    g22 = 1 - 2 * (gx * gx + gy * gy)
    G = ((g00, g01, g02), (g10, g11, g12), (g20, g21, g22))

    s0 = jnp.exp(feats_ref[4])
    s1 = jnp.exp(feats_ref[5])
    s2 = jnp.exp(feats_ref[6])
    sq = (s0 * s0, s1 * s1, s2 * s2)
    alpha = jax.nn.sigmoid(feats_ref[7])
    col_r = jnp.clip(0.5 + 0.28209479177 * feats_ref[8], 0.0, 1.0)
    col_g = jnp.clip(0.5 + 0.28209479177 * feats_ref[9], 0.0, 1.0)
    col_b = jnp.clip(0.5 + 0.28209479177 * feats_ref[10], 0.0, 1.0)

    M = [[R[a][0] * G[0][b] + R[a][1] * G[1][b] + R[a][2] * G[2][b]
          for b in range(3)] for a in range(3)]
    j00 = fx / zcl
    j02 = -fx * xc / (zcl * zcl)
    j11 = fy / zcl
    j12 = -fy * yc / (zcl * zcl)
    k0 = [j00 * M[0][b] + j02 * M[2][b] for b in range(3)]
    k1 = [j11 * M[1][b] + j12 * M[2][b] for b in range(3)]
    a = sq[0] * k0[0] * k0[0] + sq[1] * k0[1] * k0[1] + sq[2] * k0[2] * k0[2] + 0.3
    d = sq[0] * k1[0] * k1[0] + sq[1] * k1[1] * k1[1] + sq[2] * k1[2] * k1[2] + 0.3
    bb = sq[0] * k0[0] * k1[0] + sq[1] * k0[1] * k1[1] + sq[2] * k0[2] * k1[2]
    det = jnp.maximum(a * d - bb * bb, 1e-9)
    inv_a = d / det
    inv_b = -bb / det
    inv_d = a / det

    valid = ((zc > NEAR) & (zc < FAR)
             & (u >= -BT) & (u < W + BT) & (v >= -BT) & (v < H + BT)
             & (maskf_ref[0] < 0.5))
    alpha = jnp.where(valid, alpha, 0.0)

    lr = jnp.log(255.0 * jnp.maximum(alpha, 1e-12))
    lrc = jnp.maximum(lr, 0.0)
    duh = jnp.sqrt(2.0 * lrc * a) + 0.1
    dvh = jnp.sqrt(2.0 * lrc * d) + 0.1
    big = (a > 1e6) | (d > 1e6)
    duh = jnp.where(big, 1e4, duh)
    dvh = jnp.where(big, 1e4, dvh)
    txmin = jnp.clip(jnp.ceil((u - duh - (TILE - 0.5)) / TILE), 0.0, TX - 1.0)
    txmax_r = jnp.floor((u + duh - 0.5) / TILE)
    tymin = jnp.clip(jnp.ceil((v - dvh - (TILE - 0.5)) / TILE), 0.0, TY - 1.0)
    tymax_r = jnp.floor((v + dvh - 0.5) / TILE)
    hit = (valid & (lr > 0.0)
           & (u + duh >= 0.5) & (u - duh <= W - 0.5)
           & (v + dvh >= 0.5) & (v - dvh <= H - 0.5)
           & (txmax_r >= 0.0) & (tymax_r >= 0.0))
    txmax = jnp.clip(txmax_r, 0.0, TX - 1.0)
    tymax = jnp.clip(tymax_r, 0.0, TY - 1.0)
    code = jnp.where(hit,
                     txmin + 4.0 * txmax + 16.0 * tymin + 64.0 * tymax + 256.0,
                     0.0)

    zero = jnp.zeros_like(u)
    attrs_ref[0] = jnp.where(valid, u, 0.0)
    attrs_ref[1] = jnp.where(valid, v, 0.0)
    attrs_ref[2] = jnp.where(valid, inv_a, 0.0)
    attrs_ref[3] = jnp.where(valid, inv_b, 0.0)
    attrs_ref[4] = jnp.where(valid, inv_d, 0.0)
    attrs_ref[5] = alpha
    attrs_ref[6] = zero
    attrs_ref[7] = zero
    attrs_ref[8] = col_r
    attrs_ref[9] = col_g
    attrs_ref[10] = col_b
    attrs_ref[11] = zc
    attrs_ref[12] = zero + 1.0
    attrs_ref[13] = zero
    attrs_ref[14] = zero
    attrs_ref[15] = code


def _rank_body(zrow_ref, zcol_ref, rank_ref):
    zrow = zrow_ref[0:1, :]
    irow = jax.lax.broadcasted_iota(jnp.int32, (1, N), 1).astype(jnp.float32)
    acc = jnp.zeros((1, N), jnp.float32)
    for c in range(N // C):
        zc = zcol_ref[pl.ds(c * C, C), 0:1]
        jcol = (jax.lax.broadcasted_iota(jnp.int32, (C, 1), 0)
                .astype(jnp.float32) + float(c * C))
        lt = zc < zrow
        eq = (zc == zrow) & (jcol < irow)
        cmp = jnp.where(lt | eq, 1.0, 0.0)
        acc = acc + jnp.sum(cmp, axis=0, keepdims=True)
    rank_ref[0:1, :] = acc


def _perm_body(rank_hbm, codes_hbm, perm_hbm, codess_hbm,
               rank_v, codes_v, perm_v, cs_v):
    wid = lax.axis_index("s") * 2 + lax.axis_index("c")

    @pl.when(wid == 0)
    def _():
        pltpu.sync_copy(rank_hbm, rank_v)
        pltpu.sync_copy(codes_hbm, codes_v)

        def body(i, carry):
            rv = rank_v[pl.ds(i * 16, 16)]
            iv = lax.iota(jnp.int32, 16) + i * 16
            plsc.store_scatter(perm_v, [rv], iv)
            plsc.store_scatter(cs_v, [rv], codes_v[pl.ds(i * 16, 16)])
            return carry

        lax.fori_loop(0, N // 16, body, 0)
        pltpu.sync_copy(perm_v, perm_hbm)
        pltpu.sync_copy(cs_v, codess_hbm)


def _bin_body(codess_hbm, perm_hbm, attrs_hbm, binned_hbm, counts_hbm,
              codes_v, perm_v, list_v, attrs_v, rowst_v, cnt_v):
    wid = lax.axis_index("s") * 2 + lax.axis_index("c")
    t = lax.rem(wid, NT)
    h = wid // NT
    tx = lax.rem(t, TX)
    ty = t // TX

    pltpu.sync_copy(codess_hbm.at[pl.ds(h * HALF, HALF)], codes_v)
    pltpu.sync_copy(perm_hbm.at[pl.ds(h * HALF, HALF)], perm_v)
    pltpu.sync_copy(attrs_hbm, attrs_v)

    def zbody(i, carry):
        list_v[pl.ds(i * 16, 16)] = jnp.zeros((16,), jnp.int32)
        return carry

    lax.fori_loop(0, HALF // 16, zbody, 0)

    def body(i, ptr):
        ci = codes_v[pl.ds(i * 16, 16)].astype(jnp.int32)
        txmin = ci & 3
        txmax = (ci >> 2) & 3
        tymin = (ci >> 4) & 3
        tymax = (ci >> 6) & 3
        val = ci >> 8
        m = ((txmin <= tx) & (tx <= txmax) & (tymin <= ty) & (ty <= tymax)
             & (val > 0))
        plsc.store_compressed(list_v.at[pl.ds(ptr, 16)],
                              perm_v[pl.ds(i * 16, 16)], mask=m)
        return ptr + jnp.sum(m.astype(jnp.int32))

    cnt = lax.fori_loop(0, HALF // 16, body, 0)

    def gbody(j, carry):
        base = j * 16
        idxv = list_v[pl.ds(base, 16)] * 16
        for k in range(16):
            rowst_v[k, pl.ds(base, 16)] = plsc.load_gather(
                attrs_v, [idxv + k])
        return carry

    lax.fori_loop(0, (cnt + 15) // 16, gbody, 0)

    pltpu.sync_copy(rowst_v, binned_hbm.at[wid])
    cnt_v[...] = jnp.zeros((16,), jnp.int32) + cnt
    pltpu.sync_copy(cnt_v, counts_hbm.at[wid])


def _render_body(counts_ref, binned_ref, accum_ref, dep_ref):
    t = pl.program_id(0)
    txi = lax.rem(t, TX)
    tyi = t // TX
    pix = jax.lax.broadcasted_iota(jnp.int32, (TPIX, 1), 0)
    pxc = (lax.rem(pix, TILE) + txi * TILE).astype(jnp.float32) + 0.5
    pyc = (pix // TILE + tyi * TILE).astype(jnp.float32) + 0.5

    jj = jax.lax.broadcasted_iota(jnp.int32, (C, C), 0)
    kk = jax.lax.broadcasted_iota(jnp.int32, (C, C), 1)
    sut = jnp.where(jj < kk, 1.0, 0.0)

    def chunk_loop(widx, state):
        count = counts_ref[widx, 0]
        trip = (count + C - 1) // C

        def chunk(ci, st):
            accum, carry = st
            A = binned_ref[widx, :, pl.ds(ci * C, C)]
            lmask = (jax.lax.broadcasted_iota(jnp.int32, (1, C), 1)
                     < (count - ci * C))
            A = jnp.where(lmask, A, 0.0)
            u = A[0:1, :]
            v = A[1:2, :]
            ia = A[2:3, :]
            ib = A[3:4, :]
            idd = A[4:5, :]
            al = A[5:6, :]
            du = pxc - u
            dv = pyc - v
            power = -0.5 * (ia * du * du + idd * dv * dv) - ib * du * dv
            g = jnp.exp(jnp.minimum(power, 0.0))
            ai = jnp.minimum(al * g, 0.99)
            ai = jnp.where(ai > jnp.float32(1.0 / 255.0), ai, 0.0)
            logt = jnp.log(1.0 - ai)
            ecs = jnp.dot(logt, sut, preferred_element_type=jnp.float32)
            tprev = jnp.exp(carry + ecs)
            wgt = ai * tprev
            accum = accum + lax.dot_general(
                wgt, A[8:16, :], (((1,), (1,)), ((), ())),
                preferred_element_type=jnp.float32)
            carry = carry + jnp.sum(logt, axis=1, keepdims=True)
            return accum, carry

        return lax.fori_loop(0, trip, chunk, state)

    state = (jnp.zeros((TPIX, 8), jnp.float32),
             jnp.zeros((TPIX, 1), jnp.float32))
    state = chunk_loop(t, state)
    state = chunk_loop(t + NT, state)
    accum, _ = state
    accum_ref[0] = accum
    wsum = accum[:, 4:5]
    dep_ref[0] = accum[:, 3:4] / jnp.maximum(wsum, 1e-6)


@jax.jit
def _pipeline(pts3, feats11, maskf, cam):
    attrs = pl.pallas_call(
        _prep_body,
        out_shape=jax.ShapeDtypeStruct((16, 32, 128), jnp.float32),
    )(pts3, feats11, maskf, cam)
    attrs_t = attrs.reshape(16, N)
    z_row = attrs_t[11:12, :]
    z_col = z_row.T

    rank_row = pl.pallas_call(
        _rank_body,
        out_shape=jax.ShapeDtypeStruct((1, N), jnp.float32),
    )(z_row, z_col)

    rank16 = rank_row.astype(jnp.int32).reshape(N)
    codes16 = attrs_t[15, :].reshape(N)
    attrs_n = attrs_t.T

    mesh = plsc.VectorSubcoreMesh(core_axis_name="c", subcore_axis_name="s",
                                  num_cores=2, num_subcores=16)
    perm, codes_s = pl.kernel(
        _perm_body,
        out_type=[
            jax.ShapeDtypeStruct((N,), jnp.int32),
            jax.ShapeDtypeStruct((N,), jnp.float32),
        ],
        mesh=mesh,
        scratch_types=[
            pltpu.VMEM((N,), jnp.int32),
            pltpu.VMEM((N,), jnp.float32),
            pltpu.VMEM((N,), jnp.int32),
            pltpu.VMEM((N,), jnp.float32),
        ],
        compiler_params=pltpu.CompilerParams(needs_layout_passes=False),
    )(rank16, codes16)

    binned, counts = pl.kernel(
        _bin_body,
        out_type=[
            jax.ShapeDtypeStruct((NWORK, 16, HALF), jnp.float32),
            jax.ShapeDtypeStruct((NWORK, 16), jnp.int32),
        ],
        mesh=mesh,
        scratch_types=[
            pltpu.VMEM((HALF,), jnp.float32),
            pltpu.VMEM((HALF,), jnp.int32),
            pltpu.VMEM((HALF,), jnp.int32),
            pltpu.VMEM((N * 16,), jnp.float32),
            pltpu.VMEM((16, HALF), jnp.float32),
            pltpu.VMEM((16,), jnp.int32),
        ],
        compiler_params=pltpu.CompilerParams(needs_layout_passes=False),
    )(codes_s, perm, attrs_n.reshape(N * 16))

    accum, dep = pl.pallas_call(
        _render_body,
        grid=(NT,),
        in_specs=[
            pl.BlockSpec(memory_space=pltpu.SMEM),
            pl.BlockSpec((NWORK, 16, HALF), lambda t: (0, 0, 0)),
        ],
        out_specs=[
            pl.BlockSpec((1, TPIX, 8), lambda t: (t, 0, 0)),
            pl.BlockSpec((1, TPIX, 1), lambda t: (t, 0, 0)),
        ],
        out_shape=[
            jax.ShapeDtypeStruct((NT, TPIX, 8), jnp.float32),
            jax.ShapeDtypeStruct((NT, TPIX, 1), jnp.float32),
        ],
    )(counts[:, 0:1], binned)

    def detile(x):
        k = x.shape[2]
        x = x.reshape(TY, TX, TILE, TILE, k)
        x = x.transpose(0, 2, 1, 3, 4)
        return x.reshape(H, W, k)

    rgbz = detile(accum)
    img = rgbz[:, :, 0:3]
    acc = rgbz[:, :, 4]
    depth = detile(dep)[:, :, 0]
    return (img, depth, acc)


def kernel(pointcloud, pointcloud_features, point_invalid_mask,
           point_object_id, camera_intrinsics, q_camera_pointcloud,
           t_camera_pointcloud):
    del point_object_id
    pts3 = pointcloud.T.reshape(3, 32, 128)
    feat_cols = jnp.stack(
        [pointcloud_features[:, j]
         for j in (0, 1, 2, 3, 4, 5, 6, 7, 8, 24, 40)],
        axis=0).reshape(11, 32, 128)
    maskf = point_invalid_mask.astype(jnp.float32).reshape(1, 32, 128)
    cam = jnp.concatenate([
        jnp.stack([camera_intrinsics[0, 0], camera_intrinsics[1, 1],
                   camera_intrinsics[0, 2], camera_intrinsics[1, 2]]),
        q_camera_pointcloud[0],
        t_camera_pointcloud[0],
        jnp.zeros((5,), jnp.float32),
    ]).reshape(1, 16)
    return _pipeline(pts3, feat_cols, maskf, cam)


# speedup vs baseline: 19.2731x; 1.0764x over previous
import jax
import jax.numpy as jnp
from jax import lax
from jax.experimental import pallas as pl
from jax.experimental.pallas import tpu as pltpu
from jax.experimental.pallas import tpu_sc as plsc

H = 64
W = 64
N = 4096
NEAR = 0.4
FAR = 1000.0
BT = 48.0
C = 128
TILE_W = 16
TILE_H = 8
TX = W // TILE_W
TY = H // TILE_H
NT = TX * TY
TPIX = TILE_W * TILE_H
SEG = 1024


def _prep_body(pts_ref, feats_ref, maskf_ref, cam_ref, attrs_ref):
    fx = cam_ref[0, 0]
    fy = cam_ref[0, 1]
    cx = cam_ref[0, 2]
    cy = cam_ref[0, 3]
    qw = cam_ref[0, 4]
    qx = cam_ref[0, 5]
    qy = cam_ref[0, 6]
    qz = cam_ref[0, 7]
    tx = cam_ref[0, 8]
    ty = cam_ref[0, 9]
    tz = cam_ref[0, 10]
    qn = lax.rsqrt(qw * qw + qx * qx + qy * qy + qz * qz)
    w = qw * qn
    x = qx * qn
    y = qy * qn
    z_ = qz * qn
    r00 = 1 - 2 * (y * y + z_ * z_)
    r01 = 2 * (x * y - w * z_)
    r02 = 2 * (x * z_ + w * y)
    r10 = 2 * (x * y + w * z_)
    r11 = 1 - 2 * (x * x + z_ * z_)
    r12 = 2 * (y * z_ - w * x)
    r20 = 2 * (x * z_ - w * y)
    r21 = 2 * (y * z_ + w * x)
    r22 = 1 - 2 * (x * x + y * y)
    R = ((r00, r01, r02), (r10, r11, r12), (r20, r21, r22))

    px = pts_ref[0]
    py = pts_ref[1]
    pz = pts_ref[2]
    xc = r00 * px + r01 * py + r02 * pz + tx
    yc = r10 * px + r11 * py + r12 * pz + ty
    zc = r20 * px + r21 * py + r22 * pz + tz
    zcl = jnp.where(jnp.abs(zc) < 1e-6, 1e-6, zc)
    u = fx * xc / zcl + cx
    v = fy * yc / zcl + cy

    f0 = feats_ref[0]
    f1 = feats_ref[1]
    f2 = feats_ref[2]
    f3 = feats_ref[3]
    gqn = lax.rsqrt(f0 * f0 + f1 * f1 + f2 * f2 + f3 * f3)
    gw = f0 * gqn
    gx = f1 * gqn
    gy = f2 * gqn
    gz = f3 * gqn
    g00 = 1 - 2 * (gy * gy + gz * gz)
    g01 = 2 * (gx * gy - gw * gz)
    g02 = 2 * (gx * gz + gw * gy)
    g10 = 2 * (gx * gy + gw * gz)
    g11 = 1 - 2 * (gx * gx + gz * gz)
    g12 = 2 * (gy * gz - gw * gx)
    g20 = 2 * (gx * gz - gw * gy)
    g21 = 2 * (gy * gz + gw * gx)
    g22 = 1 - 2 * (gx * gx + gy * gy)
    G = ((g00, g01, g02), (g10, g11, g12), (g20, g21, g22))

    s0 = jnp.exp(feats_ref[4])
    s1 = jnp.exp(feats_ref[5])
    s2 = jnp.exp(feats_ref[6])
    sq = (s0 * s0, s1 * s1, s2 * s2)
    alpha = jax.nn.sigmoid(feats_ref[7])
    col_r = jnp.clip(0.5 + 0.28209479177 * feats_ref[8], 0.0, 1.0)
    col_g = jnp.clip(0.5 + 0.28209479177 * feats_ref[9], 0.0, 1.0)
    col_b = jnp.clip(0.5 + 0.28209479177 * feats_ref[10], 0.0, 1.0)

    M = [[R[a][0] * G[0][b] + R[a][1] * G[1][b] + R[a][2] * G[2][b]
          for b in range(3)] for a in range(3)]
    j00 = fx / zcl
    j02 = -fx * xc / (zcl * zcl)
    j11 = fy / zcl
    j12 = -fy * yc / (zcl * zcl)
    k0 = [j00 * M[0][b] + j02 * M[2][b] for b in range(3)]
    k1 = [j11 * M[1][b] + j12 * M[2][b] for b in range(3)]
    a = sq[0] * k0[0] * k0[0] + sq[1] * k0[1] * k0[1] + sq[2] * k0[2] * k0[2] + 0.3
    d = sq[0] * k1[0] * k1[0] + sq[1] * k1[1] * k1[1] + sq[2] * k1[2] * k1[2] + 0.3
    bb = sq[0] * k0[0] * k1[0] + sq[1] * k0[1] * k1[1] + sq[2] * k0[2] * k1[2]
    det = jnp.maximum(a * d - bb * bb, 1e-9)
    inv_a = d / det
    inv_b = -bb / det
    inv_d = a / det

    valid = ((zc > NEAR) & (zc < FAR)
             & (u >= -BT) & (u < W + BT) & (v >= -BT) & (v < H + BT)
             & (maskf_ref[0] < 0.5))
    alpha = jnp.where(valid, alpha, 0.0)

    lr = jnp.log(255.0 * jnp.maximum(alpha, 1e-12))
    lrc = jnp.maximum(lr, 0.0)
    duh = jnp.sqrt(2.0 * lrc * a) + 0.1
    dvh = jnp.sqrt(2.0 * lrc * d) + 0.1
    big = (a > 1e6) | (d > 1e6)
    duh = jnp.where(big, 1e4, duh)
    dvh = jnp.where(big, 1e4, dvh)
    txmin = jnp.clip(jnp.ceil((u - duh - (TILE_W - 0.5)) / TILE_W),
                     0.0, TX - 1.0)
    txmax_r = jnp.floor((u + duh - 0.5) / TILE_W)
    tymin = jnp.clip(jnp.ceil((v - dvh - (TILE_H - 0.5)) / TILE_H),
                     0.0, TY - 1.0)
    tymax_r = jnp.floor((v + dvh - 0.5) / TILE_H)
    hit = (valid & (lr > 0.0)
           & (u + duh >= 0.5) & (u - duh <= W - 0.5)
           & (v + dvh >= 0.5) & (v - dvh <= H - 0.5)
           & (txmax_r >= 0.0) & (tymax_r >= 0.0))
    txmax = jnp.clip(txmax_r, 0.0, TX - 1.0)
    tymax = jnp.clip(tymax_r, 0.0, TY - 1.0)
    code = jnp.where(hit,
                     txmin + 4.0 * txmax + 16.0 * tymin + 128.0 * tymax
                     + 1024.0,
                     0.0)

    zero = jnp.zeros_like(u)
    attrs_ref[0] = jnp.where(valid, u, 0.0)
    attrs_ref[1] = jnp.where(valid, v, 0.0)
    attrs_ref[2] = jnp.where(valid, inv_a, 0.0)
    attrs_ref[3] = jnp.where(valid, inv_b, 0.0)
    attrs_ref[4] = jnp.where(valid, inv_d, 0.0)
    attrs_ref[5] = alpha
    attrs_ref[6] = zero
    attrs_ref[7] = zero
    attrs_ref[8] = col_r
    attrs_ref[9] = col_g
    attrs_ref[10] = col_b
    attrs_ref[11] = zc
    attrs_ref[12] = zero + 1.0
    attrs_ref[13] = zero
    attrs_ref[14] = zero
    attrs_ref[15] = code


def _bin_body(codes_hbm, attrs_hbm, binned_hbm, counts_hbm,
              codes_v, attrs_v, list_v, zl_v, slist_v, rowst_v, cnt_v):
    wid = lax.axis_index("s") * 2 + lax.axis_index("c")
    tx = lax.rem(wid, TX)
    ty = wid // TX

    pltpu.sync_copy(codes_hbm, codes_v)
    pltpu.sync_copy(attrs_hbm, attrs_v)

    iota16 = lax.iota(jnp.int32, 16)

    def zbody(i, carry):
        list_v[pl.ds(i * 16, 16)] = jnp.zeros((16,), jnp.int32)
        return carry

    lax.fori_loop(0, N // 16, zbody, 0)

    def body(i, ptr):
        ci = codes_v[pl.ds(i * 16, 16)].astype(jnp.int32)
        txmin = ci & 3
        txmax = (ci >> 2) & 3
        tymin = (ci >> 4) & 7
        tymax = (ci >> 7) & 7
        val = ci >> 10
        m = ((txmin <= tx) & (tx <= txmax) & (tymin <= ty) & (ty <= tymax)
             & (val > 0))
        plsc.store_compressed(list_v.at[pl.ds(ptr, 16)], iota16 + i * 16,
                              mask=m)
        return ptr + jnp.sum(m.astype(jnp.int32))

    cnt = lax.fori_loop(0, N // 16, body, 0)
    nchunk = (cnt + 15) // 16

    def zfetch(j, carry):
        idxv = list_v[pl.ds(j * 16, 16)] * 16 + 11
        zv = plsc.load_gather(attrs_v, [idxv])
        ok = (iota16 + j * 16) < cnt
        zl_v[pl.ds(j * 16, 16)] = jnp.where(ok, zv, jnp.float32(jnp.inf))
        return carry

    lax.fori_loop(0, nchunk, zfetch, 0)

    def abody(ai, carry):
        za = zl_v[pl.ds(ai * 16, 16)]
        posa = iota16 + ai * 16

        def bbody(bi, cnta):
            base = bi * 16
            for k in range(16):
                idxk = base + ((iota16 + k) & 15)
                zb = plsc.load_gather(zl_v, [idxk])
                m = (zb < za) | ((zb == za) & (idxk < posa))
                cnta = cnta + jnp.where(m, 1, 0)
            return cnta

        cnta = lax.fori_loop(0, nchunk, bbody, jnp.zeros((16,), jnp.int32))
        plsc.store_scatter(slist_v, [cnta], list_v[pl.ds(ai * 16, 16)])
        return carry

    lax.fori_loop(0, nchunk, abody, 0)

    nseg = (cnt + SEG - 1) // SEG

    def sbody(s, carry):
        first = s * (SEG // 16)
        ntail = jnp.minimum(nchunk - first, SEG // 16)

        def gbody(j, carry2):
            base = j * 16
            idxv = slist_v[pl.ds(first * 16 + base, 16)] * 16
            for k in range(16):
                rowst_v[k, pl.ds(base, 16)] = plsc.load_gather(
                    attrs_v, [idxv + k])
            return carry2

        lax.fori_loop(0, ntail, gbody, 0)
        pltpu.sync_copy(rowst_v, binned_hbm.at[wid, :, pl.ds(s * SEG, SEG)])
        return carry

    lax.fori_loop(0, nseg, sbody, 0)

    cnt_v[...] = jnp.zeros((16,), jnp.int32) + cnt
    pltpu.sync_copy(cnt_v, counts_hbm.at[wid])


def _render_body(counts_ref, binned_ref, accum_ref, dep_ref):
    t = pl.program_id(0)
    txi = lax.rem(t, TX)
    tyi = t // TX
    pix = jax.lax.broadcasted_iota(jnp.int32, (TPIX, 1), 0)
    pxc = (lax.rem(pix, TILE_W) + txi * TILE_W).astype(jnp.float32) + 0.5
    pyc = (pix // TILE_W + tyi * TILE_H).astype(jnp.float32) + 0.5

    jj = jax.lax.broadcasted_iota(jnp.int32, (C, C), 0)
    kk = jax.lax.broadcasted_iota(jnp.int32, (C, C), 1)
    sut = jnp.where(jj < kk, 1.0, 0.0)

    count = counts_ref[t, 0]
    trip = (count + C - 1) // C

    def chunk(ci, st):
        accum, carry = st
        A = binned_ref[t, :, pl.ds(ci * C, C)]
        lmask = (jax.lax.broadcasted_iota(jnp.int32, (1, C), 1)
                 < (count - ci * C))
        A = jnp.where(lmask, A, 0.0)
        u = A[0:1, :]
        v = A[1:2, :]
        ia = A[2:3, :]
        ib = A[3:4, :]
        idd = A[4:5, :]
        al = A[5:6, :]
        du = pxc - u
        dv = pyc - v
        power = -0.5 * (ia * du * du + idd * dv * dv) - ib * du * dv
        g = jnp.exp(jnp.minimum(power, 0.0))
        ai = jnp.minimum(al * g, 0.99)
        ai = jnp.where(ai > jnp.float32(1.0 / 255.0), ai, 0.0)
        logt = jnp.log(1.0 - ai)
        ecs = jnp.dot(logt, sut, preferred_element_type=jnp.float32)
        tprev = jnp.exp(carry + ecs)
        wgt = ai * tprev
        accum = accum + lax.dot_general(
            wgt, A[8:16, :], (((1,), (1,)), ((), ())),
            preferred_element_type=jnp.float32)
        carry = carry + jnp.sum(logt, axis=1, keepdims=True)
        return accum, carry

    state = (jnp.zeros((TPIX, 8), jnp.float32),
             jnp.zeros((TPIX, 1), jnp.float32))
    accum, _ = lax.fori_loop(0, trip, chunk, state)
    accum_ref[0] = accum
    wsum = accum[:, 4:5]
    dep_ref[0] = accum[:, 3:4] / jnp.maximum(wsum, 1e-6)


@jax.jit
def _pipeline(pts3, feats11, maskf, cam):
    attrs = pl.pallas_call(
        _prep_body,
        out_shape=jax.ShapeDtypeStruct((16, 32, 128), jnp.float32),
    )(pts3, feats11, maskf, cam)
    attrs_t = attrs.reshape(16, N)
    codes = attrs_t[15, :].reshape(N)
    attrs_flat = attrs_t.T.reshape(N * 16)

    mesh = plsc.VectorSubcoreMesh(core_axis_name="c", subcore_axis_name="s",
                                  num_cores=2, num_subcores=16)
    binned, counts = pl.kernel(
        _bin_body,
        out_type=[
            jax.ShapeDtypeStruct((NT, 16, N), jnp.float32),
            jax.ShapeDtypeStruct((NT, 16), jnp.int32),
        ],
        mesh=mesh,
        scratch_types=[
            pltpu.VMEM((N,), jnp.float32),
            pltpu.VMEM((N * 16,), jnp.float32),
            pltpu.VMEM((N,), jnp.int32),
            pltpu.VMEM((N,), jnp.float32),
            pltpu.VMEM((N,), jnp.int32),
            pltpu.VMEM((16, SEG), jnp.float32),
            pltpu.VMEM((16,), jnp.int32),
        ],
        compiler_params=pltpu.CompilerParams(needs_layout_passes=False),
    )(codes, attrs_flat)

    accum, dep = pl.pallas_call(
        _render_body,
        grid=(NT,),
        in_specs=[
            pl.BlockSpec(memory_space=pltpu.SMEM),
            pl.BlockSpec((NT, 16, N), lambda t: (0, 0, 0)),
        ],
        out_specs=[
            pl.BlockSpec((1, TPIX, 8), lambda t: (t, 0, 0)),
            pl.BlockSpec((1, TPIX, 1), lambda t: (t, 0, 0)),
        ],
        out_shape=[
            jax.ShapeDtypeStruct((NT, TPIX, 8), jnp.float32),
            jax.ShapeDtypeStruct((NT, TPIX, 1), jnp.float32),
        ],
    )(counts[:, 0:1], binned)

    def detile(x):
        k = x.shape[2]
        x = x.reshape(TY, TX, TILE_H, TILE_W, k)
        x = x.transpose(0, 2, 1, 3, 4)
        return x.reshape(H, W, k)

    rgbz = detile(accum)
    img = rgbz[:, :, 0:3]
    acc = rgbz[:, :, 4]
    depth = detile(dep)[:, :, 0]
    return (img, depth, acc)


def kernel(pointcloud, pointcloud_features, point_invalid_mask,
           point_object_id, camera_intrinsics, q_camera_pointcloud,
           t_camera_pointcloud):
    del point_object_id
    pts3 = pointcloud.T.reshape(3, 32, 128)
    feat_cols = jnp.stack(
        [pointcloud_features[:, j]
         for j in (0, 1, 2, 3, 4, 5, 6, 7, 8, 24, 40)],
        axis=0).reshape(11, 32, 128)
    maskf = point_invalid_mask.astype(jnp.float32).reshape(1, 32, 128)
    cam = jnp.concatenate([
        jnp.stack([camera_intrinsics[0, 0], camera_intrinsics[1, 1],
                   camera_intrinsics[0, 2], camera_intrinsics[1, 2]]),
        q_camera_pointcloud[0],
        t_camera_pointcloud[0],
        jnp.zeros((5,), jnp.float32),
    ]).reshape(1, 16)
    return _pipeline(pts3, feat_cols, maskf, cam)


# speedup vs baseline: 20.2701x; 1.0517x over previous
import jax
import jax.numpy as jnp
from jax import lax
from jax.experimental import pallas as pl
from jax.experimental.pallas import tpu as pltpu
from jax.experimental.pallas import tpu_sc as plsc

H = 64
W = 64
N = 4096
NEAR = 0.4
FAR = 1000.0
BT = 48.0
C = 128
TILE_W = 16
TILE_H = 8
TX = W // TILE_W
TY = H // TILE_H
NT = TX * TY
TPIX = TILE_W * TILE_H
SEG = 1024


def _prep_body(pts_ref, feats_ref, maskf_ref, cam_ref, attrs_ref):
    fx = cam_ref[0, 0]
    fy = cam_ref[0, 1]
    cx = cam_ref[0, 2]
    cy = cam_ref[0, 3]
    qw = cam_ref[0, 4]
    qx = cam_ref[0, 5]
    qy = cam_ref[0, 6]
    qz = cam_ref[0, 7]
    tx = cam_ref[0, 8]
    ty = cam_ref[0, 9]
    tz = cam_ref[0, 10]
    qn = lax.rsqrt(qw * qw + qx * qx + qy * qy + qz * qz)
    w = qw * qn
    x = qx * qn
    y = qy * qn
    z_ = qz * qn
    r00 = 1 - 2 * (y * y + z_ * z_)
    r01 = 2 * (x * y - w * z_)
    r02 = 2 * (x * z_ + w * y)
    r10 = 2 * (x * y + w * z_)
    r11 = 1 - 2 * (x * x + z_ * z_)
    r12 = 2 * (y * z_ - w * x)
    r20 = 2 * (x * z_ - w * y)
    r21 = 2 * (y * z_ + w * x)
    r22 = 1 - 2 * (x * x + y * y)
    R = ((r00, r01, r02), (r10, r11, r12), (r20, r21, r22))

    px = pts_ref[0]
    py = pts_ref[1]
    pz = pts_ref[2]
    xc = r00 * px + r01 * py + r02 * pz + tx
    yc = r10 * px + r11 * py + r12 * pz + ty
    zc = r20 * px + r21 * py + r22 * pz + tz
    zcl = jnp.where(jnp.abs(zc) < 1e-6, 1e-6, zc)
    u = fx * xc / zcl + cx
    v = fy * yc / zcl + cy

    f0 = feats_ref[0]
    f1 = feats_ref[1]
    f2 = feats_ref[2]
    f3 = feats_ref[3]
    gqn = lax.rsqrt(f0 * f0 + f1 * f1 + f2 * f2 + f3 * f3)
    gw = f0 * gqn
    gx = f1 * gqn
    gy = f2 * gqn
    gz = f3 * gqn
    g00 = 1 - 2 * (gy * gy + gz * gz)
    g01 = 2 * (gx * gy - gw * gz)
    g02 = 2 * (gx * gz + gw * gy)
    g10 = 2 * (gx * gy + gw * gz)
    g11 = 1 - 2 * (gx * gx + gz * gz)
    g12 = 2 * (gy * gz - gw * gx)
    g20 = 2 * (gx * gz - gw * gy)
    g21 = 2 * (gy * gz + gw * gx)
    g22 = 1 - 2 * (gx * gx + gy * gy)
    G = ((g00, g01, g02), (g10, g11, g12), (g20, g21, g22))

    s0 = jnp.exp(feats_ref[4])
    s1 = jnp.exp(feats_ref[5])
    s2 = jnp.exp(feats_ref[6])
    sq = (s0 * s0, s1 * s1, s2 * s2)
    alpha = jax.nn.sigmoid(feats_ref[7])
    col_r = jnp.clip(0.5 + 0.28209479177 * feats_ref[8], 0.0, 1.0)
    col_g = jnp.clip(0.5 + 0.28209479177 * feats_ref[9], 0.0, 1.0)
    col_b = jnp.clip(0.5 + 0.28209479177 * feats_ref[10], 0.0, 1.0)

    M = [[R[a][0] * G[0][b] + R[a][1] * G[1][b] + R[a][2] * G[2][b]
          for b in range(3)] for a in range(3)]
    j00 = fx / zcl
    j02 = -fx * xc / (zcl * zcl)
    j11 = fy / zcl
    j12 = -fy * yc / (zcl * zcl)
    k0 = [j00 * M[0][b] + j02 * M[2][b] for b in range(3)]
    k1 = [j11 * M[1][b] + j12 * M[2][b] for b in range(3)]
    a = sq[0] * k0[0] * k0[0] + sq[1] * k0[1] * k0[1] + sq[2] * k0[2] * k0[2] + 0.3
    d = sq[0] * k1[0] * k1[0] + sq[1] * k1[1] * k1[1] + sq[2] * k1[2] * k1[2] + 0.3
    bb = sq[0] * k0[0] * k1[0] + sq[1] * k0[1] * k1[1] + sq[2] * k0[2] * k1[2]
    det = jnp.maximum(a * d - bb * bb, 1e-9)
    inv_a = d / det
    inv_b = -bb / det
    inv_d = a / det

    valid = ((zc > NEAR) & (zc < FAR)
             & (u >= -BT) & (u < W + BT) & (v >= -BT) & (v < H + BT)
             & (maskf_ref[0] < 0.5))
    alpha = jnp.where(valid, alpha, 0.0)

    lr = jnp.log(255.0 * jnp.maximum(alpha, 1e-12))
    lrc = jnp.maximum(lr, 0.0)
    duh = jnp.sqrt(2.0 * lrc * a) + 0.1
    dvh = jnp.sqrt(2.0 * lrc * d) + 0.1
    big = (a > 1e6) | (d > 1e6)
    duh = jnp.where(big, 1e4, duh)
    dvh = jnp.where(big, 1e4, dvh)
    txmin = jnp.clip(jnp.ceil((u - duh - (TILE_W - 0.5)) / TILE_W),
                     0.0, TX - 1.0)
    txmax_r = jnp.floor((u + duh - 0.5) / TILE_W)
    tymin = jnp.clip(jnp.ceil((v - dvh - (TILE_H - 0.5)) / TILE_H),
                     0.0, TY - 1.0)
    tymax_r = jnp.floor((v + dvh - 0.5) / TILE_H)
    hit = (valid & (lr > 0.0)
           & (u + duh >= 0.5) & (u - duh <= W - 0.5)
           & (v + dvh >= 0.5) & (v - dvh <= H - 0.5)
           & (txmax_r >= 0.0) & (tymax_r >= 0.0))
    txmax = jnp.clip(txmax_r, 0.0, TX - 1.0)
    tymax = jnp.clip(tymax_r, 0.0, TY - 1.0)
    code = jnp.where(hit,
                     txmin + 4.0 * txmax + 16.0 * tymin + 128.0 * tymax
                     + 1024.0,
                     0.0)

    zero = jnp.zeros_like(u)
    attrs_ref[0] = jnp.where(valid, u, 0.0)
    attrs_ref[1] = jnp.where(valid, v, 0.0)
    attrs_ref[2] = jnp.where(valid, inv_a, 0.0)
    attrs_ref[3] = jnp.where(valid, inv_b, 0.0)
    attrs_ref[4] = jnp.where(valid, inv_d, 0.0)
    attrs_ref[5] = alpha
    attrs_ref[6] = zero
    attrs_ref[7] = zero
    attrs_ref[8] = col_r
    attrs_ref[9] = col_g
    attrs_ref[10] = col_b
    attrs_ref[11] = zc
    attrs_ref[12] = zero + 1.0
    attrs_ref[13] = zero
    attrs_ref[14] = zero
    attrs_ref[15] = code


def _bin_body(codes_hbm, attrs_hbm, binned_hbm, counts_hbm,
              codes_v, attrs_v, list_v, zl_v, slist_v, rowst_v, cnt_v):
    wid = lax.axis_index("s") * 2 + lax.axis_index("c")
    tx = lax.rem(wid, TX)
    ty = wid // TX

    pltpu.sync_copy(codes_hbm, codes_v)
    pltpu.sync_copy(attrs_hbm, attrs_v)

    iota16 = lax.iota(jnp.int32, 16)

    def zbody(i, carry):
        list_v[pl.ds(i * 16, 16)] = jnp.zeros((16,), jnp.int32)
        return carry

    lax.fori_loop(0, N // 16, zbody, 0)

    def body(i, ptr):
        ci = codes_v[pl.ds(i * 16, 16)].astype(jnp.int32)
        txmin = ci & 3
        txmax = (ci >> 2) & 3
        tymin = (ci >> 4) & 7
        tymax = (ci >> 7) & 7
        val = ci >> 10
        m = ((txmin <= tx) & (tx <= txmax) & (tymin <= ty) & (ty <= tymax)
             & (val > 0))
        plsc.store_compressed(list_v.at[pl.ds(ptr, 16)], iota16 + i * 16,
                              mask=m)
        return ptr + jnp.sum(m.astype(jnp.int32))

    cnt = lax.fori_loop(0, N // 16, body, 0)
    nchunk = (cnt + 15) // 16

    def zfetch(j, carry):
        idxv = list_v[pl.ds(j * 16, 16)] + 11 * N
        zv = plsc.load_gather(attrs_v, [idxv])
        ok = (iota16 + j * 16) < cnt
        zl_v[pl.ds(j * 16, 16)] = jnp.where(ok, zv, jnp.float32(jnp.inf))
        return carry

    lax.fori_loop(0, nchunk, zfetch, 0)

    def abody(ai, carry):
        za = zl_v[pl.ds(ai * 16, 16)]

        def bbody(bi, cnta):
            base = bi * 16

            def off_le(c):
                for k in range(16):
                    zb = plsc.load_gather(zl_v, [base + ((iota16 + k) & 15)])
                    c = c + jnp.where(zb <= za, 1, 0)
                return c

            def off_lt(c):
                for k in range(16):
                    zb = plsc.load_gather(zl_v, [base + ((iota16 + k) & 15)])
                    c = c + jnp.where(zb < za, 1, 0)
                return c

            def diag(c):
                for k in range(16):
                    rot = (iota16 + k) & 15
                    zb = plsc.load_gather(zl_v, [base + rot])
                    m = (zb < za) | ((zb == za) & (rot < iota16))
                    c = c + jnp.where(m, 1, 0)
                return c

            return lax.cond(bi < ai, off_le,
                            lambda c: lax.cond(bi == ai, diag, off_lt, c),
                            cnta)

        cnta = lax.fori_loop(0, nchunk, bbody, jnp.zeros((16,), jnp.int32))
        plsc.store_scatter(slist_v, [cnta], list_v[pl.ds(ai * 16, 16)])
        return carry

    lax.fori_loop(0, nchunk, abody, 0)

    nseg = (cnt + SEG - 1) // SEG

    def sbody(s, carry):
        first = s * (SEG // 16)
        ntail = jnp.minimum(nchunk - first, SEG // 16)

        def gbody(j, carry2):
            base = j * 16
            idxv = slist_v[pl.ds(first * 16 + base, 16)]
            for k in range(16):
                rowst_v[k, pl.ds(base, 16)] = plsc.load_gather(
                    attrs_v, [idxv + k * N])
            return carry2

        lax.fori_loop(0, ntail, gbody, 0)
        pltpu.sync_copy(rowst_v, binned_hbm.at[wid, :, pl.ds(s * SEG, SEG)])
        return carry

    lax.fori_loop(0, nseg, sbody, 0)

    cnt_v[...] = jnp.zeros((16,), jnp.int32) + cnt
    pltpu.sync_copy(cnt_v, counts_hbm.at[wid])


def _render_body(counts_ref, binned_ref, accum_ref, dep_ref):
    t = pl.program_id(0)
    txi = lax.rem(t, TX)
    tyi = t // TX
    pix = jax.lax.broadcasted_iota(jnp.int32, (TPIX, 1), 0)
    pxc = (lax.rem(pix, TILE_W) + txi * TILE_W).astype(jnp.float32) + 0.5
    pyc = (pix // TILE_W + tyi * TILE_H).astype(jnp.float32) + 0.5

    jj = jax.lax.broadcasted_iota(jnp.int32, (C, C), 0)
    kk = jax.lax.broadcasted_iota(jnp.int32, (C, C), 1)
    sut = jnp.where(jj < kk, 1.0, 0.0)

    count = counts_ref[t, 0]
    trip = (count + C - 1) // C

    def chunk(ci, st):
        accum, carry = st
        A = binned_ref[0, :, pl.ds(ci * C, C)]
        lmask = (jax.lax.broadcasted_iota(jnp.int32, (1, C), 1)
                 < (count - ci * C))
        A = jnp.where(lmask, A, 0.0)
        u = A[0:1, :]
        v = A[1:2, :]
        ia = A[2:3, :]
        ib = A[3:4, :]
        idd = A[4:5, :]
        al = A[5:6, :]
        du = pxc - u
        dv = pyc - v
        power = -0.5 * (ia * du * du + idd * dv * dv) - ib * du * dv
        g = jnp.exp(jnp.minimum(power, 0.0))
        ai = jnp.minimum(al * g, 0.99)
        ai = jnp.where(ai > jnp.float32(1.0 / 255.0), ai, 0.0)
        logt = jnp.log(1.0 - ai)
        ecs = jnp.dot(logt, sut, preferred_element_type=jnp.float32)
        tprev = jnp.exp(carry + ecs)
        wgt = ai * tprev
        accum = accum + lax.dot_general(
            wgt, A[8:16, :], (((1,), (1,)), ((), ())),
            preferred_element_type=jnp.float32)
        carry = carry + jnp.sum(logt, axis=1, keepdims=True)
        return accum, carry

    state = (jnp.zeros((TPIX, 8), jnp.float32),
             jnp.zeros((TPIX, 1), jnp.float32))
    accum, _ = lax.fori_loop(0, trip, chunk, state)
    accum_ref[0] = accum
    wsum = accum[:, 4:5]
    dep_ref[0] = accum[:, 3:4] / jnp.maximum(wsum, 1e-6)


@jax.jit
def _pipeline(pts3, feats11, maskf, cam):
    attrs = pl.pallas_call(
        _prep_body,
        out_shape=jax.ShapeDtypeStruct((16, 32, 128), jnp.float32),
    )(pts3, feats11, maskf, cam)
    attrs_t = attrs.reshape(16, N)
    codes = attrs_t[15, :].reshape(N)
    attrs_flat = attrs_t.reshape(16 * N)

    mesh = plsc.VectorSubcoreMesh(core_axis_name="c", subcore_axis_name="s",
                                  num_cores=2, num_subcores=16)
    binned, counts = pl.kernel(
        _bin_body,
        out_type=[
            jax.ShapeDtypeStruct((NT, 16, N), jnp.float32),
            jax.ShapeDtypeStruct((NT, 16), jnp.int32),
        ],
        mesh=mesh,
        scratch_types=[
            pltpu.VMEM((N,), jnp.float32),
            pltpu.VMEM((N * 16,), jnp.float32),
            pltpu.VMEM((N,), jnp.int32),
            pltpu.VMEM((N,), jnp.float32),
            pltpu.VMEM((N,), jnp.int32),
            pltpu.VMEM((16, SEG), jnp.float32),
            pltpu.VMEM((16,), jnp.int32),
        ],
        compiler_params=pltpu.CompilerParams(needs_layout_passes=False),
    )(codes, attrs_flat)

    accum, dep = pl.pallas_call(
        _render_body,
        grid=(NT,),
        in_specs=[
            pl.BlockSpec(memory_space=pltpu.SMEM),
            pl.BlockSpec((1, 16, N), lambda t: (t, 0, 0)),
        ],
        out_specs=[
            pl.BlockSpec((1, TPIX, 8), lambda t: (t, 0, 0)),
            pl.BlockSpec((1, TPIX, 1), lambda t: (t, 0, 0)),
        ],
        out_shape=[
            jax.ShapeDtypeStruct((NT, TPIX, 8), jnp.float32),
            jax.ShapeDtypeStruct((NT, TPIX, 1), jnp.float32),
        ],
    )(counts[:, 0:1], binned)

    def detile(x):
        k = x.shape[2]
        x = x.reshape(TY, TX, TILE_H, TILE_W, k)
        x = x.transpose(0, 2, 1, 3, 4)
        return x.reshape(H, W, k)

    rgbz = detile(accum)
    img = rgbz[:, :, 0:3]
    acc = rgbz[:, :, 4]
    depth = detile(dep)[:, :, 0]
    return (img, depth, acc)


def kernel(pointcloud, pointcloud_features, point_invalid_mask,
           point_object_id, camera_intrinsics, q_camera_pointcloud,
           t_camera_pointcloud):
    del point_object_id
    pts3 = pointcloud.T.reshape(3, 32, 128)
    feat_cols = jnp.stack(
        [pointcloud_features[:, j]
         for j in (0, 1, 2, 3, 4, 5, 6, 7, 8, 24, 40)],
        axis=0).reshape(11, 32, 128)
    maskf = point_invalid_mask.astype(jnp.float32).reshape(1, 32, 128)
    cam = jnp.concatenate([
        jnp.stack([camera_intrinsics[0, 0], camera_intrinsics[1, 1],
                   camera_intrinsics[0, 2], camera_intrinsics[1, 2]]),
        q_camera_pointcloud[0],
        t_camera_pointcloud[0],
        jnp.zeros((5,), jnp.float32),
    ]).reshape(1, 16)
    return _pipeline(pts3, feat_cols, maskf, cam)


# speedup vs baseline: 20.3212x; 1.0025x over previous
import jax
import jax.numpy as jnp
from jax import lax
from jax.experimental import pallas as pl
from jax.experimental.pallas import tpu as pltpu
from jax.experimental.pallas import tpu_sc as plsc

H = 64
W = 64
N = 4096
NEAR = 0.4
FAR = 1000.0
BT = 48.0
C = 128
TILE_W = 16
TILE_H = 8
TX = W // TILE_W
TY = H // TILE_H
NT = TX * TY
TPIX = TILE_W * TILE_H
SEG = 1024


def _prep_body(pts_ref, feats_ref, maskf_ref, cam_ref, attrs_ref):
    fx = cam_ref[0, 0]
    fy = cam_ref[0, 1]
    cx = cam_ref[0, 2]
    cy = cam_ref[0, 3]
    qw = cam_ref[0, 4]
    qx = cam_ref[0, 5]
    qy = cam_ref[0, 6]
    qz = cam_ref[0, 7]
    tx = cam_ref[0, 8]
    ty = cam_ref[0, 9]
    tz = cam_ref[0, 10]
    qn = lax.rsqrt(qw * qw + qx * qx + qy * qy + qz * qz)
    w = qw * qn
    x = qx * qn
    y = qy * qn
    z_ = qz * qn
    r00 = 1 - 2 * (y * y + z_ * z_)
    r01 = 2 * (x * y - w * z_)
    r02 = 2 * (x * z_ + w * y)
    r10 = 2 * (x * y + w * z_)
    r11 = 1 - 2 * (x * x + z_ * z_)
    r12 = 2 * (y * z_ - w * x)
    r20 = 2 * (x * z_ - w * y)
    r21 = 2 * (y * z_ + w * x)
    r22 = 1 - 2 * (x * x + y * y)
    R = ((r00, r01, r02), (r10, r11, r12), (r20, r21, r22))

    px = pts_ref[0]
    py = pts_ref[1]
    pz = pts_ref[2]
    xc = r00 * px + r01 * py + r02 * pz + tx
    yc = r10 * px + r11 * py + r12 * pz + ty
    zc = r20 * px + r21 * py + r22 * pz + tz
    zcl = jnp.where(jnp.abs(zc) < 1e-6, 1e-6, zc)
    u = fx * xc / zcl + cx
    v = fy * yc / zcl + cy

    f0 = feats_ref[0]
    f1 = feats_ref[1]
    f2 = feats_ref[2]
    f3 = feats_ref[3]
    gqn = lax.rsqrt(f0 * f0 + f1 * f1 + f2 * f2 + f3 * f3)
    gw = f0 * gqn
    gx = f1 * gqn
    gy = f2 * gqn
    gz = f3 * gqn
    g00 = 1 - 2 * (gy * gy + gz * gz)
    g01 = 2 * (gx * gy - gw * gz)
    g02 = 2 * (gx * gz + gw * gy)
    g10 = 2 * (gx * gy + gw * gz)
    g11 = 1 - 2 * (gx * gx + gz * gz)
    g12 = 2 * (gy * gz - gw * gx)
    g20 = 2 * (gx * gz - gw * gy)
    g21 = 2 * (gy * gz + gw * gx)
    g22 = 1 - 2 * (gx * gx + gy * gy)
    G = ((g00, g01, g02), (g10, g11, g12), (g20, g21, g22))

    s0 = jnp.exp(feats_ref[4])
    s1 = jnp.exp(feats_ref[5])
    s2 = jnp.exp(feats_ref[6])
    sq = (s0 * s0, s1 * s1, s2 * s2)
    alpha = jax.nn.sigmoid(feats_ref[7])
    col_r = jnp.clip(0.5 + 0.28209479177 * feats_ref[8], 0.0, 1.0)
    col_g = jnp.clip(0.5 + 0.28209479177 * feats_ref[9], 0.0, 1.0)
    col_b = jnp.clip(0.5 + 0.28209479177 * feats_ref[10], 0.0, 1.0)

    M = [[R[a][0] * G[0][b] + R[a][1] * G[1][b] + R[a][2] * G[2][b]
          for b in range(3)] for a in range(3)]
    j00 = fx / zcl
    j02 = -fx * xc / (zcl * zcl)
    j11 = fy / zcl
    j12 = -fy * yc / (zcl * zcl)
    k0 = [j00 * M[0][b] + j02 * M[2][b] for b in range(3)]
    k1 = [j11 * M[1][b] + j12 * M[2][b] for b in range(3)]
    a = sq[0] * k0[0] * k0[0] + sq[1] * k0[1] * k0[1] + sq[2] * k0[2] * k0[2] + 0.3
    d = sq[0] * k1[0] * k1[0] + sq[1] * k1[1] * k1[1] + sq[2] * k1[2] * k1[2] + 0.3
    bb = sq[0] * k0[0] * k1[0] + sq[1] * k0[1] * k1[1] + sq[2] * k0[2] * k1[2]
    det = jnp.maximum(a * d - bb * bb, 1e-9)
    inv_a = d / det
    inv_b = -bb / det
    inv_d = a / det

    valid = ((zc > NEAR) & (zc < FAR)
             & (u >= -BT) & (u < W + BT) & (v >= -BT) & (v < H + BT)
             & (maskf_ref[0] < 0.5))
    alpha = jnp.where(valid, alpha, 0.0)

    lr = jnp.log(255.0 * jnp.maximum(alpha, 1e-12))
    lrc = jnp.maximum(lr, 0.0)
    duh = jnp.sqrt(2.0 * lrc * a) + 0.1
    dvh = jnp.sqrt(2.0 * lrc * d) + 0.1
    big = (a > 1e6) | (d > 1e6)
    duh = jnp.where(big, 1e4, duh)
    dvh = jnp.where(big, 1e4, dvh)
    txmin = jnp.clip(jnp.ceil((u - duh - (TILE_W - 0.5)) / TILE_W),
                     0.0, TX - 1.0)
    txmax_r = jnp.floor((u + duh - 0.5) / TILE_W)
    tymin = jnp.clip(jnp.ceil((v - dvh - (TILE_H - 0.5)) / TILE_H),
                     0.0, TY - 1.0)
    tymax_r = jnp.floor((v + dvh - 0.5) / TILE_H)
    hit = (valid & (lr > 0.0)
           & (u + duh >= 0.5) & (u - duh <= W - 0.5)
           & (v + dvh >= 0.5) & (v - dvh <= H - 0.5)
           & (txmax_r >= 0.0) & (tymax_r >= 0.0))
    txmax = jnp.clip(txmax_r, 0.0, TX - 1.0)
    tymax = jnp.clip(tymax_r, 0.0, TY - 1.0)
    code = jnp.where(hit,
                     txmin + 4.0 * txmax + 16.0 * tymin + 128.0 * tymax
                     + 1024.0,
                     0.0)

    zero = jnp.zeros_like(u)
    attrs_ref[0] = jnp.where(valid, u, 0.0)
    attrs_ref[1] = jnp.where(valid, v, 0.0)
    attrs_ref[2] = jnp.where(valid, inv_a, 0.0)
    attrs_ref[3] = jnp.where(valid, inv_b, 0.0)
    attrs_ref[4] = jnp.where(valid, inv_d, 0.0)
    attrs_ref[5] = alpha
    attrs_ref[6] = zero
    attrs_ref[7] = zero
    attrs_ref[8] = col_r
    attrs_ref[9] = col_g
    attrs_ref[10] = col_b
    attrs_ref[11] = zc
    attrs_ref[12] = zero + 1.0
    attrs_ref[13] = zero
    attrs_ref[14] = zero
    attrs_ref[15] = code


def _bin_body(codes_hbm, attrs_hbm, binned_hbm, counts_hbm,
              codes_v, attrs_v, list_v, zl_v, slist_v, rowst_v, cnt_v):
    wid = lax.axis_index("s") * 2 + lax.axis_index("c")
    tx = lax.rem(wid, TX)
    ty = wid // TX

    pltpu.sync_copy(codes_hbm, codes_v)
    pltpu.sync_copy(attrs_hbm, attrs_v)

    iota16 = lax.iota(jnp.int32, 16)

    def zbody(i, carry):
        list_v[pl.ds(i * 16, 16)] = jnp.zeros((16,), jnp.int32)
        return carry

    lax.fori_loop(0, N // 16, zbody, 0)

    def body(i, ptr):
        ci = codes_v[pl.ds(i * 16, 16)].astype(jnp.int32)
        txmin = ci & 3
        txmax = (ci >> 2) & 3
        tymin = (ci >> 4) & 7
        tymax = (ci >> 7) & 7
        val = ci >> 10
        m = ((txmin <= tx) & (tx <= txmax) & (tymin <= ty) & (ty <= tymax)
             & (val > 0))
        plsc.store_compressed(list_v.at[pl.ds(ptr, 16)], iota16 + i * 16,
                              mask=m)
        return ptr + jnp.sum(m.astype(jnp.int32))

    cnt = lax.fori_loop(0, N // 16, body, 0)
    nchunk = (cnt + 15) // 16

    def zfetch(j, carry):
        idxv = list_v[pl.ds(j * 16, 16)] + 11 * N
        zv = plsc.load_gather(attrs_v, [idxv])
        ok = (iota16 + j * 16) < cnt
        zl_v[pl.ds(j * 16, 16)] = jnp.where(ok, zv, jnp.float32(jnp.inf))
        return carry

    lax.fori_loop(0, nchunk, zfetch, 0)

    def abody(ai, carry):
        za = zl_v[pl.ds(ai * 16, 16)]

        def bbody(bi, cnta):
            base = bi * 16

            def off_le(c):
                for k in range(16):
                    zb = plsc.load_gather(zl_v, [base + ((iota16 + k) & 15)])
                    c = c + jnp.where(zb <= za, 1, 0)
                return c

            def off_lt(c):
                for k in range(16):
                    zb = plsc.load_gather(zl_v, [base + ((iota16 + k) & 15)])
                    c = c + jnp.where(zb < za, 1, 0)
                return c

            def diag(c):
                for k in range(16):
                    rot = (iota16 + k) & 15
                    zb = plsc.load_gather(zl_v, [base + rot])
                    m = (zb < za) | ((zb == za) & (rot < iota16))
                    c = c + jnp.where(m, 1, 0)
                return c

            return lax.cond(bi < ai, off_le,
                            lambda c: lax.cond(bi == ai, diag, off_lt, c),
                            cnta)

        cnta = lax.fori_loop(0, nchunk, bbody, jnp.zeros((16,), jnp.int32))
        plsc.store_scatter(slist_v, [cnta], list_v[pl.ds(ai * 16, 16)])
        return carry

    lax.fori_loop(0, nchunk, abody, 0)

    nseg = (cnt + SEG - 1) // SEG

    def sbody(s, carry):
        first = s * (SEG // 16)
        ntail = jnp.minimum(nchunk - first, SEG // 16)

        def gbody(j, carry2):
            base = j * 16
            idxv = slist_v[pl.ds(first * 16 + base, 16)]
            for k in range(16):
                rowst_v[k, pl.ds(base, 16)] = plsc.load_gather(
                    attrs_v, [idxv + k * N])
            return carry2

        lax.fori_loop(0, ntail, gbody, 0)
        pltpu.sync_copy(rowst_v, binned_hbm.at[wid, :, pl.ds(s * SEG, SEG)])
        return carry

    lax.fori_loop(0, nseg, sbody, 0)

    cnt_v[...] = jnp.zeros((16,), jnp.int32) + cnt
    pltpu.sync_copy(cnt_v, counts_hbm.at[wid])


def _render_body(counts_ref, binned_ref, accum_ref, dep_ref):
    t = pl.program_id(0)
    txi = lax.rem(t, TX)
    tyi = t // TX
    pix = jax.lax.broadcasted_iota(jnp.int32, (TPIX, 1), 0)
    pxc = (lax.rem(pix, TILE_W) + txi * TILE_W).astype(jnp.float32) + 0.5
    pyc = (pix // TILE_W + tyi * TILE_H).astype(jnp.float32) + 0.5

    jj = jax.lax.broadcasted_iota(jnp.int32, (C, C), 0)
    kk = jax.lax.broadcasted_iota(jnp.int32, (C, C), 1)
    sut = jnp.where(jj < kk, 1.0, 0.0)

    count = counts_ref[t, 0]
    trip = (count + C - 1) // C

    def chunk(ci, st):
        accum, carry = st
        A = binned_ref[0, :, pl.ds(ci * C, C)]
        lmask = (jax.lax.broadcasted_iota(jnp.int32, (1, C), 1)
                 < (count - ci * C))
        A = jnp.where(lmask, A, 0.0)
        u = A[0:1, :]
        v = A[1:2, :]
        ia = A[2:3, :]
        ib = A[3:4, :]
        idd = A[4:5, :]
        al = A[5:6, :]
        du = pxc - u
        dv = pyc - v
        power = -0.5 * (ia * du * du + idd * dv * dv) - ib * du * dv
        g = jnp.exp(jnp.minimum(power, 0.0))
        ai = jnp.minimum(al * g, 0.99)
        ai = jnp.where(ai > jnp.float32(1.0 / 255.0), ai, 0.0)
        logt = jnp.log(1.0 - ai)
        ecs = jnp.dot(logt, sut, preferred_element_type=jnp.float32)
        tprev = jnp.exp(carry + ecs)
        wgt = ai * tprev
        accum = accum + lax.dot_general(
            wgt, A[8:16, :], (((1,), (1,)), ((), ())),
            preferred_element_type=jnp.float32)
        carry = carry + jnp.sum(logt, axis=1, keepdims=True)
        return accum, carry

    state = (jnp.zeros((TPIX, 8), jnp.float32),
             jnp.zeros((TPIX, 1), jnp.float32))
    accum, _ = lax.fori_loop(0, trip, chunk, state)
    accum_ref[0] = accum
    wsum = accum[:, 4:5]
    dep_ref[0] = accum[:, 3:4] / jnp.maximum(wsum, 1e-6)


@jax.jit
def _pipeline(pts3, feats11, maskf, cam):
    attrs = pl.pallas_call(
        _prep_body,
        out_shape=jax.ShapeDtypeStruct((16, 32, 128), jnp.float32),
    )(pts3, feats11, maskf, cam)
    attrs_t = attrs.reshape(16, N)
    codes = attrs_t[15, :].reshape(N)
    attrs_flat = attrs_t.reshape(16 * N)

    mesh = plsc.VectorSubcoreMesh(core_axis_name="c", subcore_axis_name="s",
                                  num_cores=2, num_subcores=16)
    binned, counts = pl.kernel(
        _bin_body,
        out_type=[
            jax.ShapeDtypeStruct((NT, 16, N), jnp.float32),
            jax.ShapeDtypeStruct((NT, 16), jnp.int32),
        ],
        mesh=mesh,
        scratch_types=[
            pltpu.VMEM((N,), jnp.float32),
            pltpu.VMEM((N * 16,), jnp.float32),
            pltpu.VMEM((N,), jnp.int32),
            pltpu.VMEM((N,), jnp.float32),
            pltpu.VMEM((N,), jnp.int32),
            pltpu.VMEM((16, SEG), jnp.float32),
            pltpu.VMEM((16,), jnp.int32),
        ],
        compiler_params=pltpu.CompilerParams(needs_layout_passes=False),
    )(codes, attrs_flat)

    accum, dep = pl.pallas_call(
        _render_body,
        grid=(NT,),
        in_specs=[
            pl.BlockSpec(memory_space=pltpu.SMEM),
            pl.BlockSpec((1, 16, N), lambda t: (t, 0, 0)),
        ],
        out_specs=[
            pl.BlockSpec((1, TPIX, 8), lambda t: (t, 0, 0)),
            pl.BlockSpec((1, TPIX, 1), lambda t: (t, 0, 0)),
        ],
        out_shape=[
            jax.ShapeDtypeStruct((NT, TPIX, 8), jnp.float32),
            jax.ShapeDtypeStruct((NT, TPIX, 1), jnp.float32),
        ],
    )(counts[:, 0:1], binned)

    def detile(x):
        k = x.shape[2]
        x = x.reshape(TY, TX, TILE_H, TILE_W, k)
        x = x.transpose(0, 2, 1, 3, 4)
        return x.reshape(H, W, k)

    rgbz = detile(accum)
    img = rgbz[:, :, 0:3]
    acc = rgbz[:, :, 4]
    depth = detile(dep)[:, :, 0]
    return (img, depth, acc)


def kernel(pointcloud, pointcloud_features, point_invalid_mask,
           point_object_id, camera_intrinsics, q_camera_pointcloud,
           t_camera_pointcloud):
    del point_object_id
    pts3 = pointcloud.T.reshape(3, 32, 128)
    feats_t = pointcloud_features.T
    feat_cols = jnp.concatenate(
        [feats_t[0:9], feats_t[24:25], feats_t[40:41]],
        axis=0).reshape(11, 32, 128)
    maskf = point_invalid_mask.astype(jnp.float32).reshape(1, 32, 128)
    cam = jnp.concatenate([
        jnp.stack([camera_intrinsics[0, 0], camera_intrinsics[1, 1],
                   camera_intrinsics[0, 2], camera_intrinsics[1, 2]]),
        q_camera_pointcloud[0],
        t_camera_pointcloud[0],
        jnp.zeros((5,), jnp.float32),
    ]).reshape(1, 16)
    return _pipeline(pts3, feat_cols, maskf, cam)


# speedup vs baseline: 21.0351x; 1.0351x over previous
import jax
import jax.numpy as jnp
from jax import lax
from jax.experimental import pallas as pl
from jax.experimental.pallas import tpu as pltpu
from jax.experimental.pallas import tpu_sc as plsc

H = 64
W = 64
N = 4096
NEAR = 0.4
FAR = 1000.0
BT = 48.0
C = 128
TILE_W = 16
TILE_H = 8
TX = W // TILE_W
TY = H // TILE_H
NT = TX * TY
TPIX = TILE_W * TILE_H
SEG = 1024


def _prep_body(pts_ref, feats_ref, maskf_ref, cam_ref, attrs_ref):
    fx = cam_ref[0, 0]
    fy = cam_ref[0, 1]
    cx = cam_ref[0, 2]
    cy = cam_ref[0, 3]
    qw = cam_ref[0, 4]
    qx = cam_ref[0, 5]
    qy = cam_ref[0, 6]
    qz = cam_ref[0, 7]
    tx = cam_ref[0, 8]
    ty = cam_ref[0, 9]
    tz = cam_ref[0, 10]
    qn = lax.rsqrt(qw * qw + qx * qx + qy * qy + qz * qz)
    w = qw * qn
    x = qx * qn
    y = qy * qn
    z_ = qz * qn
    r00 = 1 - 2 * (y * y + z_ * z_)
    r01 = 2 * (x * y - w * z_)
    r02 = 2 * (x * z_ + w * y)
    r10 = 2 * (x * y + w * z_)
    r11 = 1 - 2 * (x * x + z_ * z_)
    r12 = 2 * (y * z_ - w * x)
    r20 = 2 * (x * z_ - w * y)
    r21 = 2 * (y * z_ + w * x)
    r22 = 1 - 2 * (x * x + y * y)
    R = ((r00, r01, r02), (r10, r11, r12), (r20, r21, r22))

    px = pts_ref[0]
    py = pts_ref[1]
    pz = pts_ref[2]
    xc = r00 * px + r01 * py + r02 * pz + tx
    yc = r10 * px + r11 * py + r12 * pz + ty
    zc = r20 * px + r21 * py + r22 * pz + tz
    zcl = jnp.where(jnp.abs(zc) < 1e-6, 1e-6, zc)
    u = fx * xc / zcl + cx
    v = fy * yc / zcl + cy

    f0 = feats_ref[0]
    f1 = feats_ref[1]
    f2 = feats_ref[2]
    f3 = feats_ref[3]
    gqn = lax.rsqrt(f0 * f0 + f1 * f1 + f2 * f2 + f3 * f3)
    gw = f0 * gqn
    gx = f1 * gqn
    gy = f2 * gqn
    gz = f3 * gqn
    g00 = 1 - 2 * (gy * gy + gz * gz)
    g01 = 2 * (gx * gy - gw * gz)
    g02 = 2 * (gx * gz + gw * gy)
    g10 = 2 * (gx * gy + gw * gz)
    g11 = 1 - 2 * (gx * gx + gz * gz)
    g12 = 2 * (gy * gz - gw * gx)
    g20 = 2 * (gx * gz - gw * gy)
    g21 = 2 * (gy * gz + gw * gx)
    g22 = 1 - 2 * (gx * gx + gy * gy)
    G = ((g00, g01, g02), (g10, g11, g12), (g20, g21, g22))

    s0 = jnp.exp(feats_ref[4])
    s1 = jnp.exp(feats_ref[5])
    s2 = jnp.exp(feats_ref[6])
    sq = (s0 * s0, s1 * s1, s2 * s2)
    alpha = jax.nn.sigmoid(feats_ref[7])
    col_r = jnp.clip(0.5 + 0.28209479177 * feats_ref[8], 0.0, 1.0)
    col_g = jnp.clip(0.5 + 0.28209479177 * feats_ref[9], 0.0, 1.0)
    col_b = jnp.clip(0.5 + 0.28209479177 * feats_ref[10], 0.0, 1.0)

    M = [[R[a][0] * G[0][b] + R[a][1] * G[1][b] + R[a][2] * G[2][b]
          for b in range(3)] for a in range(3)]
    j00 = fx / zcl
    j02 = -fx * xc / (zcl * zcl)
    j11 = fy / zcl
    j12 = -fy * yc / (zcl * zcl)
    k0 = [j00 * M[0][b] + j02 * M[2][b] for b in range(3)]
    k1 = [j11 * M[1][b] + j12 * M[2][b] for b in range(3)]
    a = sq[0] * k0[0] * k0[0] + sq[1] * k0[1] * k0[1] + sq[2] * k0[2] * k0[2] + 0.3
    d = sq[0] * k1[0] * k1[0] + sq[1] * k1[1] * k1[1] + sq[2] * k1[2] * k1[2] + 0.3
    bb = sq[0] * k0[0] * k1[0] + sq[1] * k0[1] * k1[1] + sq[2] * k0[2] * k1[2]
    det = jnp.maximum(a * d - bb * bb, 1e-9)
    inv_a = d / det
    inv_b = -bb / det
    inv_d = a / det

    valid = ((zc > NEAR) & (zc < FAR)
             & (u >= -BT) & (u < W + BT) & (v >= -BT) & (v < H + BT)
             & (maskf_ref[0] < 0.5))
    alpha = jnp.where(valid, alpha, 0.0)

    lr = jnp.log(255.0 * jnp.maximum(alpha, 1e-12))
    lrc = jnp.maximum(lr, 0.0)
    duh = jnp.sqrt(2.0 * lrc * a) + 0.1
    dvh = jnp.sqrt(2.0 * lrc * d) + 0.1
    big = (a > 1e6) | (d > 1e6)
    duh = jnp.where(big, 1e4, duh)
    dvh = jnp.where(big, 1e4, dvh)
    txmin = jnp.clip(jnp.ceil((u - duh - (TILE_W - 0.5)) / TILE_W),
                     0.0, TX - 1.0)
    txmax_r = jnp.floor((u + duh - 0.5) / TILE_W)
    tymin = jnp.clip(jnp.ceil((v - dvh - (TILE_H - 0.5)) / TILE_H),
                     0.0, TY - 1.0)
    tymax_r = jnp.floor((v + dvh - 0.5) / TILE_H)
    hit = (valid & (lr > 0.0)
           & (u + duh >= 0.5) & (u - duh <= W - 0.5)
           & (v + dvh >= 0.5) & (v - dvh <= H - 0.5)
           & (txmax_r >= 0.0) & (tymax_r >= 0.0))
    txmax = jnp.clip(txmax_r, 0.0, TX - 1.0)
    tymax = jnp.clip(tymax_r, 0.0, TY - 1.0)
    code = jnp.where(hit,
                     txmin + 4.0 * txmax + 16.0 * tymin + 128.0 * tymax
                     + 1024.0,
                     0.0)

    zero = jnp.zeros_like(u)
    attrs_ref[0] = jnp.where(valid, u, 0.0)
    attrs_ref[1] = jnp.where(valid, v, 0.0)
    attrs_ref[2] = jnp.where(valid, inv_a, 0.0)
    attrs_ref[3] = jnp.where(valid, inv_b, 0.0)
    attrs_ref[4] = jnp.where(valid, inv_d, 0.0)
    attrs_ref[5] = alpha
    attrs_ref[6] = zero
    attrs_ref[7] = zero
    attrs_ref[8] = col_r
    attrs_ref[9] = col_g
    attrs_ref[10] = col_b
    attrs_ref[11] = zc
    attrs_ref[12] = zero + 1.0
    attrs_ref[13] = zero
    attrs_ref[14] = zero
    attrs_ref[15] = code


def _bin_body(attrs_hbm, binned_hbm, counts_hbm,
              attrs_v, list_v, zl_v, slist_v, rowst_v, cnt_v):
    wid = lax.axis_index("s") * 2 + lax.axis_index("c")
    tx = lax.rem(wid, TX)
    ty = wid // TX

    pltpu.sync_copy(attrs_hbm, attrs_v)

    iota16 = lax.iota(jnp.int32, 16)

    def zbody(i, carry):
        list_v[pl.ds(i * 16, 16)] = jnp.zeros((16,), jnp.int32)
        return carry

    lax.fori_loop(0, N // 16, zbody, 0)

    def body(i, ptr):
        ci = attrs_v[pl.ds(15 * N + i * 16, 16)].astype(jnp.int32)
        txmin = ci & 3
        txmax = (ci >> 2) & 3
        tymin = (ci >> 4) & 7
        tymax = (ci >> 7) & 7
        val = ci >> 10
        m = ((txmin <= tx) & (tx <= txmax) & (tymin <= ty) & (ty <= tymax)
             & (val > 0))
        plsc.store_compressed(list_v.at[pl.ds(ptr, 16)], iota16 + i * 16,
                              mask=m)
        return ptr + jnp.sum(m.astype(jnp.int32))

    cnt = lax.fori_loop(0, N // 16, body, 0)
    nchunk = (cnt + 15) // 16

    def zfetch(j, carry):
        idxv = list_v[pl.ds(j * 16, 16)] + 11 * N
        zv = plsc.load_gather(attrs_v, [idxv])
        ok = (iota16 + j * 16) < cnt
        zl_v[pl.ds(j * 16, 16)] = jnp.where(ok, zv, jnp.float32(jnp.inf))
        return carry

    lax.fori_loop(0, nchunk, zfetch, 0)

    def abody(ai, carry):
        za = zl_v[pl.ds(ai * 16, 16)]

        def bbody(bi, cnta):
            base = bi * 16

            def off_le(c):
                for k in range(16):
                    zb = plsc.load_gather(zl_v, [base + ((iota16 + k) & 15)])
                    c = c + jnp.where(zb <= za, 1, 0)
                return c

            def off_lt(c):
                for k in range(16):
                    zb = plsc.load_gather(zl_v, [base + ((iota16 + k) & 15)])
                    c = c + jnp.where(zb < za, 1, 0)
                return c

            def diag(c):
                for k in range(16):
                    rot = (iota16 + k) & 15
                    zb = plsc.load_gather(zl_v, [base + rot])
                    m = (zb < za) | ((zb == za) & (rot < iota16))
                    c = c + jnp.where(m, 1, 0)
                return c

            return lax.cond(bi < ai, off_le,
                            lambda c: lax.cond(bi == ai, diag, off_lt, c),
                            cnta)

        cnta = lax.fori_loop(0, nchunk, bbody, jnp.zeros((16,), jnp.int32))
        plsc.store_scatter(slist_v, [cnta], list_v[pl.ds(ai * 16, 16)])
        return carry

    lax.fori_loop(0, nchunk, abody, 0)

    nseg = (cnt + SEG - 1) // SEG

    def sbody(s, carry):
        first = s * (SEG // 16)
        ntail = jnp.minimum(nchunk - first, SEG // 16)

        def gbody(j, carry2):
            base = j * 16
            idxv = slist_v[pl.ds(first * 16 + base, 16)]
            for k in range(16):
                rowst_v[k, pl.ds(base, 16)] = plsc.load_gather(
                    attrs_v, [idxv + k * N])
            return carry2

        lax.fori_loop(0, ntail, gbody, 0)
        pltpu.sync_copy(rowst_v, binned_hbm.at[wid, :, pl.ds(s * SEG, SEG)])
        return carry

    lax.fori_loop(0, nseg, sbody, 0)

    cnt_v[...] = jnp.zeros((16,), jnp.int32) + cnt
    pltpu.sync_copy(cnt_v, counts_hbm.at[wid])


def _render_body(counts_ref, binned_ref, accum_ref, dep_ref):
    s = pl.program_id(0)

    jj = jax.lax.broadcasted_iota(jnp.int32, (C, C), 0)
    kk = jax.lax.broadcasted_iota(jnp.int32, (C, C), 1)
    sut = jnp.where(jj < kk, 1.0, 0.0)
    pix = jax.lax.broadcasted_iota(jnp.int32, (TPIX, 1), 0)

    for sub in range(2):
        t = s * 2 + sub
        txi = lax.rem(t, TX)
        tyi = t // TX
        pxc = (lax.rem(pix, TILE_W) + txi * TILE_W).astype(jnp.float32) + 0.5
        pyc = (pix // TILE_W + tyi * TILE_H).astype(jnp.float32) + 0.5

        count = counts_ref[t, 0]
        trip = (count + C - 1) // C

        def chunk(ci, st):
            accum, carry = st
            A = binned_ref[sub, :, pl.ds(ci * C, C)]
            lmask = (jax.lax.broadcasted_iota(jnp.int32, (1, C), 1)
                     < (count - ci * C))
            A = jnp.where(lmask, A, 0.0)
            u = A[0:1, :]
            v = A[1:2, :]
            ia = A[2:3, :]
            ib = A[3:4, :]
            idd = A[4:5, :]
            al = A[5:6, :]
            du = pxc - u
            dv = pyc - v
            power = -0.5 * (ia * du * du + idd * dv * dv) - ib * du * dv
            g = jnp.exp(jnp.minimum(power, 0.0))
            ai = jnp.minimum(al * g, 0.99)
            ai = jnp.where(ai > jnp.float32(1.0 / 255.0), ai, 0.0)
            logt = jnp.log(1.0 - ai)
            ecs = jnp.dot(logt, sut, preferred_element_type=jnp.float32)
            tprev = jnp.exp(carry + ecs)
            wgt = ai * tprev
            accum = accum + lax.dot_general(
                wgt, A[8:16, :], (((1,), (1,)), ((), ())),
                preferred_element_type=jnp.float32)
            carry = carry + jnp.sum(logt, axis=1, keepdims=True)
            return accum, carry

        state = (jnp.zeros((TPIX, 8), jnp.float32),
                 jnp.zeros((TPIX, 1), jnp.float32))
        accum, _ = lax.fori_loop(0, trip, chunk, state)
        accum_ref[sub] = accum
        wsum = accum[:, 4:5]
        dep_ref[sub] = accum[:, 3:4] / jnp.maximum(wsum, 1e-6)


@jax.jit
def _pipeline(pts3, feats11, maskf, cam):
    attrs = pl.pallas_call(
        _prep_body,
        out_shape=jax.ShapeDtypeStruct((16, 32, 128), jnp.float32),
    )(pts3, feats11, maskf, cam)
    attrs_flat = attrs.reshape(16 * N)

    mesh = plsc.VectorSubcoreMesh(core_axis_name="c", subcore_axis_name="s",
                                  num_cores=2, num_subcores=16)
    binned, counts = pl.kernel(
        _bin_body,
        out_type=[
            jax.ShapeDtypeStruct((NT, 16, N), jnp.float32),
            jax.ShapeDtypeStruct((NT, 16), jnp.int32),
        ],
        mesh=mesh,
        scratch_types=[
            pltpu.VMEM((N * 16,), jnp.float32),
            pltpu.VMEM((N,), jnp.int32),
            pltpu.VMEM((N,), jnp.float32),
            pltpu.VMEM((N,), jnp.int32),
            pltpu.VMEM((16, SEG), jnp.float32),
            pltpu.VMEM((16,), jnp.int32),
        ],
        compiler_params=pltpu.CompilerParams(needs_layout_passes=False),
    )(attrs_flat)

    accum, dep = pl.pallas_call(
        _render_body,
        grid=(NT // 2,),
        in_specs=[
            pl.BlockSpec(memory_space=pltpu.SMEM),
            pl.BlockSpec((2, 16, N), lambda t: (t, 0, 0)),
        ],
        out_specs=[
            pl.BlockSpec((2, TPIX, 8), lambda t: (t, 0, 0)),
            pl.BlockSpec((2, TPIX, 1), lambda t: (t, 0, 0)),
        ],
        out_shape=[
            jax.ShapeDtypeStruct((NT, TPIX, 8), jnp.float32),
            jax.ShapeDtypeStruct((NT, TPIX, 1), jnp.float32),
        ],
    )(counts[:, 0:1], binned)

    def detile(x):
        k = x.shape[2]
        x = x.reshape(TY, TX, TILE_H, TILE_W, k)
        x = x.transpose(0, 2, 1, 3, 4)
        return x.reshape(H, W, k)

    rgbz = detile(accum)
    img = rgbz[:, :, 0:3]
    acc = rgbz[:, :, 4]
    depth = detile(dep)[:, :, 0]
    return (img, depth, acc)


def kernel(pointcloud, pointcloud_features, point_invalid_mask,
           point_object_id, camera_intrinsics, q_camera_pointcloud,
           t_camera_pointcloud):
    del point_object_id
    pts3 = pointcloud.T.reshape(3, 32, 128)
    feats_t = pointcloud_features.T
    feat_cols = jnp.concatenate(
        [feats_t[0:9], feats_t[24:25], feats_t[40:41]],
        axis=0).reshape(11, 32, 128)
    maskf = point_invalid_mask.astype(jnp.float32).reshape(1, 32, 128)
    cam = jnp.concatenate([
        jnp.stack([camera_intrinsics[0, 0], camera_intrinsics[1, 1],
                   camera_intrinsics[0, 2], camera_intrinsics[1, 2]]),
        q_camera_pointcloud[0],
        t_camera_pointcloud[0],
        jnp.zeros((5,), jnp.float32),
    ]).reshape(1, 16)
    return _pipeline(pts3, feat_cols, maskf, cam)


# speedup vs baseline: 22.5072x; 1.0700x over previous
import jax
import jax.numpy as jnp
from jax import lax
from jax.experimental import pallas as pl
from jax.experimental.pallas import tpu as pltpu
from jax.experimental.pallas import tpu_sc as plsc

H = 64
W = 64
N = 4096
NEAR = 0.4
FAR = 1000.0
BT = 48.0
C = 128
TILE_W = 16
TILE_H = 8
TX = W // TILE_W
TY = H // TILE_H
NT = TX * TY
TPIX = TILE_W * TILE_H
SEG = 1024


def _prep_body(pts_ref, feats_ref, maskf_ref, cam_ref, attrs_ref):
    fx = cam_ref[0, 0]
    fy = cam_ref[0, 1]
    cx = cam_ref[0, 2]
    cy = cam_ref[0, 3]
    qw = cam_ref[0, 4]
    qx = cam_ref[0, 5]
    qy = cam_ref[0, 6]
    qz = cam_ref[0, 7]
    tx = cam_ref[0, 8]
    ty = cam_ref[0, 9]
    tz = cam_ref[0, 10]
    qn = lax.rsqrt(qw * qw + qx * qx + qy * qy + qz * qz)
    w = qw * qn
    x = qx * qn
    y = qy * qn
    z_ = qz * qn
    r00 = 1 - 2 * (y * y + z_ * z_)
    r01 = 2 * (x * y - w * z_)
    r02 = 2 * (x * z_ + w * y)
    r10 = 2 * (x * y + w * z_)
    r11 = 1 - 2 * (x * x + z_ * z_)
    r12 = 2 * (y * z_ - w * x)
    r20 = 2 * (x * z_ - w * y)
    r21 = 2 * (y * z_ + w * x)
    r22 = 1 - 2 * (x * x + y * y)
    R = ((r00, r01, r02), (r10, r11, r12), (r20, r21, r22))

    px = pts_ref[0]
    py = pts_ref[1]
    pz = pts_ref[2]
    xc = r00 * px + r01 * py + r02 * pz + tx
    yc = r10 * px + r11 * py + r12 * pz + ty
    zc = r20 * px + r21 * py + r22 * pz + tz
    zcl = jnp.where(jnp.abs(zc) < 1e-6, 1e-6, zc)
    u = fx * xc / zcl + cx
    v = fy * yc / zcl + cy

    f0 = feats_ref[0]
    f1 = feats_ref[1]
    f2 = feats_ref[2]
    f3 = feats_ref[3]
    gqn = lax.rsqrt(f0 * f0 + f1 * f1 + f2 * f2 + f3 * f3)
    gw = f0 * gqn
    gx = f1 * gqn
    gy = f2 * gqn
    gz = f3 * gqn
    g00 = 1 - 2 * (gy * gy + gz * gz)
    g01 = 2 * (gx * gy - gw * gz)
    g02 = 2 * (gx * gz + gw * gy)
    g10 = 2 * (gx * gy + gw * gz)
    g11 = 1 - 2 * (gx * gx + gz * gz)
    g12 = 2 * (gy * gz - gw * gx)
    g20 = 2 * (gx * gz - gw * gy)
    g21 = 2 * (gy * gz + gw * gx)
    g22 = 1 - 2 * (gx * gx + gy * gy)
    G = ((g00, g01, g02), (g10, g11, g12), (g20, g21, g22))

    s0 = jnp.exp(feats_ref[4])
    s1 = jnp.exp(feats_ref[5])
    s2 = jnp.exp(feats_ref[6])
    sq = (s0 * s0, s1 * s1, s2 * s2)
    alpha = jax.nn.sigmoid(feats_ref[7])
    col_r = jnp.clip(0.5 + 0.28209479177 * feats_ref[8], 0.0, 1.0)
    col_g = jnp.clip(0.5 + 0.28209479177 * feats_ref[9], 0.0, 1.0)
    col_b = jnp.clip(0.5 + 0.28209479177 * feats_ref[10], 0.0, 1.0)

    M = [[R[a][0] * G[0][b] + R[a][1] * G[1][b] + R[a][2] * G[2][b]
          for b in range(3)] for a in range(3)]
    j00 = fx / zcl
    j02 = -fx * xc / (zcl * zcl)
    j11 = fy / zcl
    j12 = -fy * yc / (zcl * zcl)
    k0 = [j00 * M[0][b] + j02 * M[2][b] for b in range(3)]
    k1 = [j11 * M[1][b] + j12 * M[2][b] for b in range(3)]
    a = sq[0] * k0[0] * k0[0] + sq[1] * k0[1] * k0[1] + sq[2] * k0[2] * k0[2] + 0.3
    d = sq[0] * k1[0] * k1[0] + sq[1] * k1[1] * k1[1] + sq[2] * k1[2] * k1[2] + 0.3
    bb = sq[0] * k0[0] * k1[0] + sq[1] * k0[1] * k1[1] + sq[2] * k0[2] * k1[2]
    det = jnp.maximum(a * d - bb * bb, 1e-9)
    inv_a = d / det
    inv_b = -bb / det
    inv_d = a / det

    valid = ((zc > NEAR) & (zc < FAR)
             & (u >= -BT) & (u < W + BT) & (v >= -BT) & (v < H + BT)
             & (maskf_ref[0] < 0.5))
    alpha = jnp.where(valid, alpha, 0.0)

    lr = jnp.log(255.0 * jnp.maximum(alpha, 1e-12))
    lrc = jnp.maximum(lr, 0.0)
    duh = jnp.sqrt(2.0 * lrc * a) + 0.1
    dvh = jnp.sqrt(2.0 * lrc * d) + 0.1
    big = (a > 1e6) | (d > 1e6)
    duh = jnp.where(big, 1e4, duh)
    dvh = jnp.where(big, 1e4, dvh)
    txmin = jnp.clip(jnp.ceil((u - duh - (TILE_W - 0.5)) / TILE_W),
                     0.0, TX - 1.0)
    txmax_r = jnp.floor((u + duh - 0.5) / TILE_W)
    tymin = jnp.clip(jnp.ceil((v - dvh - (TILE_H - 0.5)) / TILE_H),
                     0.0, TY - 1.0)
    tymax_r = jnp.floor((v + dvh - 0.5) / TILE_H)
    hit = (valid & (lr > 0.0)
           & (u + duh >= 0.5) & (u - duh <= W - 0.5)
           & (v + dvh >= 0.5) & (v - dvh <= H - 0.5)
           & (txmax_r >= 0.0) & (tymax_r >= 0.0))
    txmax = jnp.clip(txmax_r, 0.0, TX - 1.0)
    tymax = jnp.clip(tymax_r, 0.0, TY - 1.0)
    code = jnp.where(hit,
                     txmin + 4.0 * txmax + 16.0 * tymin + 128.0 * tymax
                     + 1024.0,
                     0.0)

    zero = jnp.zeros_like(u)
    attrs_ref[0] = jnp.where(valid, u, 0.0)
    attrs_ref[1] = jnp.where(valid, v, 0.0)
    attrs_ref[2] = jnp.where(valid, inv_a, 0.0)
    attrs_ref[3] = jnp.where(valid, inv_b, 0.0)
    attrs_ref[4] = jnp.where(valid, inv_d, 0.0)
    attrs_ref[5] = alpha
    attrs_ref[6] = zero
    attrs_ref[7] = zero
    attrs_ref[8] = col_r
    attrs_ref[9] = col_g
    attrs_ref[10] = col_b
    attrs_ref[11] = zc
    attrs_ref[12] = zero + 1.0
    attrs_ref[13] = zero
    attrs_ref[14] = zero
    attrs_ref[15] = code


def _bin_body(attrs_hbm, binned_hbm, counts_hbm,
              attrs_v, codes_v, zf_v, list_v, zl_v, slist_v, rowst_v, cnt_v,
              sem):
    wid = lax.axis_index("s") * 2 + lax.axis_index("c")
    tx = lax.rem(wid, TX)
    ty = wid // TX

    pltpu.sync_copy(attrs_hbm.at[pl.ds(11 * N, N)], zf_v)
    pltpu.sync_copy(attrs_hbm.at[pl.ds(15 * N, N)], codes_v)
    full_cp = pltpu.async_copy(attrs_hbm, attrs_v, sem)

    iota16 = lax.iota(jnp.int32, 16)

    def zbody(i, carry):
        list_v[pl.ds(i * 16, 16)] = jnp.zeros((16,), jnp.int32)
        return carry

    lax.fori_loop(0, N // 16, zbody, 0)

    def body(i, ptr):
        ci = codes_v[pl.ds(i * 16, 16)].astype(jnp.int32)
        txmin = ci & 3
        txmax = (ci >> 2) & 3
        tymin = (ci >> 4) & 7
        tymax = (ci >> 7) & 7
        val = ci >> 10
        m = ((txmin <= tx) & (tx <= txmax) & (tymin <= ty) & (ty <= tymax)
             & (val > 0))
        plsc.store_compressed(list_v.at[pl.ds(ptr, 16)], iota16 + i * 16,
                              mask=m)
        return ptr + jnp.sum(m.astype(jnp.int32))

    cnt = lax.fori_loop(0, N // 16, body, 0)
    nchunk = (cnt + 15) // 16

    def zfetch(j, carry):
        idxv = list_v[pl.ds(j * 16, 16)]
        zv = plsc.load_gather(zf_v, [idxv])
        ok = (iota16 + j * 16) < cnt
        zl_v[pl.ds(j * 16, 16)] = jnp.where(ok, zv, jnp.float32(jnp.inf))
        return carry

    lax.fori_loop(0, nchunk, zfetch, 0)

    def abody(ai, carry):
        za = zl_v[pl.ds(ai * 16, 16)]

        def bbody(bi, cnta):
            base = bi * 16

            def off_le(c):
                for k in range(16):
                    zb = plsc.load_gather(zl_v, [base + ((iota16 + k) & 15)])
                    c = c + jnp.where(zb <= za, 1, 0)
                return c

            def off_lt(c):
                for k in range(16):
                    zb = plsc.load_gather(zl_v, [base + ((iota16 + k) & 15)])
                    c = c + jnp.where(zb < za, 1, 0)
                return c

            def diag(c):
                for k in range(16):
                    rot = (iota16 + k) & 15
                    zb = plsc.load_gather(zl_v, [base + rot])
                    m = (zb < za) | ((zb == za) & (rot < iota16))
                    c = c + jnp.where(m, 1, 0)
                return c

            return lax.cond(bi < ai, off_le,
                            lambda c: lax.cond(bi == ai, diag, off_lt, c),
                            cnta)

        cnta = lax.fori_loop(0, nchunk, bbody, jnp.zeros((16,), jnp.int32))
        plsc.store_scatter(slist_v, [cnta], list_v[pl.ds(ai * 16, 16)])
        return carry

    lax.fori_loop(0, nchunk, abody, 0)

    full_cp.wait()
    nseg = (cnt + SEG - 1) // SEG

    def sbody(s, carry):
        first = s * (SEG // 16)
        ntail = jnp.minimum(nchunk - first, SEG // 16)

        def gbody(j, carry2):
            base = j * 16
            idxv = slist_v[pl.ds(first * 16 + base, 16)]
            for k in range(16):
                rowst_v[k, pl.ds(base, 16)] = plsc.load_gather(
                    attrs_v, [idxv + k * N])
            return carry2

        lax.fori_loop(0, ntail, gbody, 0)
        pltpu.sync_copy(rowst_v, binned_hbm.at[wid, :, pl.ds(s * SEG, SEG)])
        return carry

    lax.fori_loop(0, nseg, sbody, 0)

    cnt_v[...] = jnp.zeros((16,), jnp.int32) + cnt
    pltpu.sync_copy(cnt_v, counts_hbm.at[wid])


def _render_body(counts_ref, binned_ref, accum_ref, dep_ref):
    s = pl.program_id(0)

    jj = jax.lax.broadcasted_iota(jnp.int32, (C, C), 0)
    kk = jax.lax.broadcasted_iota(jnp.int32, (C, C), 1)
    sut = jnp.where(jj < kk, 1.0, 0.0)
    pix = jax.lax.broadcasted_iota(jnp.int32, (TPIX, 1), 0)

    for sub in range(2):
        t = s * 2 + sub
        txi = lax.rem(t, TX)
        tyi = t // TX
        pxc = (lax.rem(pix, TILE_W) + txi * TILE_W).astype(jnp.float32) + 0.5
        pyc = (pix // TILE_W + tyi * TILE_H).astype(jnp.float32) + 0.5

        count = counts_ref[t, 0]
        trip = (count + C - 1) // C

        def chunk(ci, st):
            accum, carry = st
            A = binned_ref[sub, :, pl.ds(ci * C, C)]
            lmask = (jax.lax.broadcasted_iota(jnp.int32, (1, C), 1)
                     < (count - ci * C))
            A = jnp.where(lmask, A, 0.0)
            u = A[0:1, :]
            v = A[1:2, :]
            ia = A[2:3, :]
            ib = A[3:4, :]
            idd = A[4:5, :]
            al = A[5:6, :]
            du = pxc - u
            dv = pyc - v
            power = -0.5 * (ia * du * du + idd * dv * dv) - ib * du * dv
            g = jnp.exp(jnp.minimum(power, 0.0))
            ai = jnp.minimum(al * g, 0.99)
            ai = jnp.where(ai > jnp.float32(1.0 / 255.0), ai, 0.0)
            logt = jnp.log(1.0 - ai)
            ecs = jnp.dot(logt, sut, preferred_element_type=jnp.float32)
            tprev = jnp.exp(carry + ecs)
            wgt = ai * tprev
            accum = accum + lax.dot_general(
                wgt, A[8:16, :], (((1,), (1,)), ((), ())),
                preferred_element_type=jnp.float32)
            carry = carry + jnp.sum(logt, axis=1, keepdims=True)
            return accum, carry

        state = (jnp.zeros((TPIX, 8), jnp.float32),
                 jnp.zeros((TPIX, 1), jnp.float32))
        accum, _ = lax.fori_loop(0, trip, chunk, state)
        accum_ref[sub] = accum
        wsum = accum[:, 4:5]
        dep_ref[sub] = accum[:, 3:4] / jnp.maximum(wsum, 1e-6)


@jax.jit
def _pipeline(pts3, feats11, maskf, cam):
    attrs = pl.pallas_call(
        _prep_body,
        out_shape=jax.ShapeDtypeStruct((16, 32, 128), jnp.float32),
    )(pts3, feats11, maskf, cam)
    attrs_flat = attrs.reshape(16 * N)

    mesh = plsc.VectorSubcoreMesh(core_axis_name="c", subcore_axis_name="s",
                                  num_cores=2, num_subcores=16)
    binned, counts = pl.kernel(
        _bin_body,
        out_type=[
            jax.ShapeDtypeStruct((NT, 16, N), jnp.float32),
            jax.ShapeDtypeStruct((NT, 16), jnp.int32),
        ],
        mesh=mesh,
        scratch_types=[
            pltpu.VMEM((N * 16,), jnp.float32),
            pltpu.VMEM((N,), jnp.float32),
            pltpu.VMEM((N,), jnp.float32),
            pltpu.VMEM((N,), jnp.int32),
            pltpu.VMEM((N,), jnp.float32),
            pltpu.VMEM((N,), jnp.int32),
            pltpu.VMEM((16, SEG), jnp.float32),
            pltpu.VMEM((16,), jnp.int32),
            pltpu.SemaphoreType.DMA,
        ],
        compiler_params=pltpu.CompilerParams(needs_layout_passes=False),
    )(attrs_flat)

    accum, dep = pl.pallas_call(
        _render_body,
        grid=(NT // 2,),
        in_specs=[
            pl.BlockSpec(memory_space=pltpu.SMEM),
            pl.BlockSpec((2, 16, N), lambda t: (t, 0, 0)),
        ],
        out_specs=[
            pl.BlockSpec((2, TPIX, 8), lambda t: (t, 0, 0)),
            pl.BlockSpec((2, TPIX, 1), lambda t: (t, 0, 0)),
        ],
        out_shape=[
            jax.ShapeDtypeStruct((NT, TPIX, 8), jnp.float32),
            jax.ShapeDtypeStruct((NT, TPIX, 1), jnp.float32),
        ],
    )(counts[:, 0:1], binned)

    def detile(x):
        k = x.shape[2]
        x = x.reshape(TY, TX, TILE_H, TILE_W, k)
        x = x.transpose(0, 2, 1, 3, 4)
        return x.reshape(H, W, k)

    rgbz = detile(accum)
    img = rgbz[:, :, 0:3]
    acc = rgbz[:, :, 4]
    depth = detile(dep)[:, :, 0]
    return (img, depth, acc)


def kernel(pointcloud, pointcloud_features, point_invalid_mask,
           point_object_id, camera_intrinsics, q_camera_pointcloud,
           t_camera_pointcloud):
    del point_object_id
    pts3 = pointcloud.T.reshape(3, 32, 128)
    feats_t = pointcloud_features.T
    feat_cols = jnp.concatenate(
        [feats_t[0:9], feats_t[24:25], feats_t[40:41]],
        axis=0).reshape(11, 32, 128)
    maskf = point_invalid_mask.astype(jnp.float32).reshape(1, 32, 128)
    cam = jnp.concatenate([
        jnp.stack([camera_intrinsics[0, 0], camera_intrinsics[1, 1],
                   camera_intrinsics[0, 2], camera_intrinsics[1, 2]]),
        q_camera_pointcloud[0],
        t_camera_pointcloud[0],
        jnp.zeros((5,), jnp.float32),
    ]).reshape(1, 16)
    return _pipeline(pts3, feat_cols, maskf, cam)
